# Initial kernel scaffold; baseline (speedup 1.0000x reference)
import functools

import jax
import jax.numpy as jnp
from jax import lax
from jax.experimental import pallas as pl
from jax.experimental.pallas import tpu as pltpu
from jax.experimental.pallas import tpu_sc as plsc

NC = 2
NS = 16
NW = NC * NS
LANES = 16
CHUNK = 80

_INV_SCALE = 1.0 / (128.0 ** 0.5)


def _mesh():
    return plsc.VectorSubcoreMesh(
        core_axis_name="c", subcore_axis_name="s", num_cores=NC, num_subcores=NS)


def _wid():
    return lax.axis_index("s") * NC + lax.axis_index("c")


def _sc_attn(q, k, v, src, dst, zeros_nd, n, e):
    ep = e // NW
    nch = ep // CHUNK
    assert ep % CHUNK == 0 and n % NS == 0
    rps = n // NS

    def body(q_hbm, k_hbm, v_hbm, src_hbm, dst_hbm, zf_hbm,
             z_out, den_out, cnt_out,
             srcv, dstv, qbuf, kbuf, vbuf, den_l, cnt_l, z_sp):
        cid = lax.axis_index("c")
        sid = lax.axis_index("s")
        wid = _wid()

        def zloc(i, c):
            zv = jnp.zeros((LANES,), jnp.float32)
            den_l[pl.ds(i * LANES, LANES)] = zv
            cnt_l[pl.ds(i * LANES, LANES)] = zv
            return c
        lax.fori_loop(0, n // LANES, zloc, 0)

        pltpu.sync_copy(zf_hbm.at[pl.ds(sid * rps, rps)],
                        z_sp.at[pl.ds(sid * rps, rps)])
        plsc.subcore_barrier()

        ones16 = jnp.ones((LANES,), jnp.float32)

        def chunk_body(ci, carry):
            base = wid * ep + ci * CHUNK
            pltpu.sync_copy(src_hbm.at[pl.ds(base, CHUNK)], srcv)
            pltpu.sync_copy(dst_hbm.at[pl.ds(base, CHUNK)], dstv)
            pltpu.sync_copy(q_hbm.at[dstv], qbuf)
            pltpu.sync_copy(k_hbm.at[srcv], kbuf)
            pltpu.sync_copy(v_hbm.at[srcv], vbuf)
            for g in range(CHUNK // LANES):
                rows = g * LANES + lax.iota(jnp.int32, LANES)
                dstg = dstv[pl.ds(g * LANES, LANES)]

                def dot_body(dd, acc):
                    for u in range(4):
                        col = jnp.full((LANES,), dd * 4 + u, jnp.int32)
                        qv = plsc.load_gather(qbuf, [rows, col])
                        kv = plsc.load_gather(kbuf, [rows, col])
                        acc = acc + qv * kv
                    return acc
                acc = lax.fori_loop(0, 32, dot_body,
                                    jnp.zeros((LANES,), jnp.float32))
                w = jnp.exp(acc * _INV_SCALE)
                plsc.addupdate_scatter(den_l, [dstg], w)
                plsc.addupdate_scatter(cnt_l, [dstg], ones16)

                def scale_body(dd, c):
                    for u in range(4):
                        col = jnp.full((LANES,), dd * 4 + u, jnp.int32)
                        vv = plsc.load_gather(vbuf, [rows, col])
                        plsc.store_scatter(vbuf, [rows, col], vv * w)
                    return c
                lax.fori_loop(0, 32, scale_body, 0)
            pltpu.sync_copy(vbuf, z_sp.at[dstv], add=True)
            return carry
        lax.fori_loop(0, nch, chunk_body, 0)

        pltpu.sync_copy(den_l, den_out.at[wid])
        pltpu.sync_copy(cnt_l, cnt_out.at[wid])
        plsc.subcore_barrier()
        pltpu.sync_copy(z_sp.at[pl.ds(sid * rps, rps)],
                        z_out.at[cid, pl.ds(sid * rps, rps)])

    fn = pl.kernel(
        body,
        out_type=(jax.ShapeDtypeStruct((NC, n, 128), jnp.float32),
                  jax.ShapeDtypeStruct((NW, n), jnp.float32),
                  jax.ShapeDtypeStruct((NW, n), jnp.float32)),
        mesh=_mesh(),
        scratch_types=[
            pltpu.VMEM((CHUNK,), jnp.int32),
            pltpu.VMEM((CHUNK,), jnp.int32),
            pltpu.VMEM((CHUNK, 128), jnp.float32),
            pltpu.VMEM((CHUNK, 128), jnp.float32),
            pltpu.VMEM((CHUNK, 128), jnp.float32),
            pltpu.VMEM((n,), jnp.float32),
            pltpu.VMEM((n,), jnp.float32),
            pltpu.VMEM_SHARED((n, 128), jnp.float32),
        ],
    )
    return fn(q, k, v, src, dst, zeros_nd)


def _sc_sage(h, src, dst, zeros_nd, n, e):
    ep = e // NW
    nch = ep // CHUNK
    rps = n // NS

    def body(h_hbm, src_hbm, dst_hbm, zf_hbm, s_out,
             srcv, dstv, buf, s_sp):
        cid = lax.axis_index("c")
        sid = lax.axis_index("s")
        wid = _wid()
        pltpu.sync_copy(zf_hbm.at[pl.ds(sid * rps, rps)],
                        s_sp.at[pl.ds(sid * rps, rps)])
        plsc.subcore_barrier()

        def chunk_body(ci, carry):
            base = wid * ep + ci * CHUNK
            pltpu.sync_copy(src_hbm.at[pl.ds(base, CHUNK)], srcv)
            pltpu.sync_copy(dst_hbm.at[pl.ds(base, CHUNK)], dstv)
            pltpu.sync_copy(h_hbm.at[srcv], buf)
            pltpu.sync_copy(buf, s_sp.at[dstv], add=True)
            return carry
        lax.fori_loop(0, nch, chunk_body, 0)

        plsc.subcore_barrier()
        pltpu.sync_copy(s_sp.at[pl.ds(sid * rps, rps)],
                        s_out.at[cid, pl.ds(sid * rps, rps)])

    fn = pl.kernel(
        body,
        out_type=jax.ShapeDtypeStruct((NC, n, 128), jnp.float32),
        mesh=_mesh(),
        scratch_types=[
            pltpu.VMEM((CHUNK,), jnp.int32),
            pltpu.VMEM((CHUNK,), jnp.int32),
            pltpu.VMEM((CHUNK, 128), jnp.float32),
            pltpu.VMEM_SHARED((n, 128), jnp.float32),
        ],
    )
    return fn(h, src, dst, zeros_nd)


_ROWS = 2000


def _tc_qkvs(x, wc, bc, n):
    grid = n // _ROWS

    def body(x_ref, w_ref, b_ref, o_ref):
        o_ref[...] = jnp.dot(x_ref[...], w_ref[...],
                             preferred_element_type=jnp.float32) + b_ref[...]

    return pl.pallas_call(
        body,
        grid=(grid,),
        in_specs=[
            pl.BlockSpec((_ROWS, 128), lambda i: (i, 0)),
            pl.BlockSpec((128, 512), lambda i: (0, 0)),
            pl.BlockSpec((1, 512), lambda i: (0, 0)),
        ],
        out_specs=pl.BlockSpec((_ROWS, 512), lambda i: (i, 0)),
        out_shape=jax.ShapeDtypeStruct((n, 512), jnp.float32),
    )(x, wc, bc)


def _tc_h(zp, denp, s, n):
    grid = n // _ROWS

    def body(z_ref, d_ref, s_ref, o_ref):
        z = z_ref[0] + z_ref[1]
        den = jnp.sum(d_ref[...], axis=0) + 1e-16
        o_ref[...] = jnp.maximum(z / den[:, None] + s_ref[...], 0.0)

    return pl.pallas_call(
        body,
        grid=(grid,),
        in_specs=[
            pl.BlockSpec((NC, _ROWS, 128), lambda i: (0, i, 0)),
            pl.BlockSpec((NW, _ROWS), lambda i: (0, i)),
            pl.BlockSpec((_ROWS, 128), lambda i: (i, 0)),
        ],
        out_specs=pl.BlockSpec((_ROWS, 128), lambda i: (i, 0)),
        out_shape=jax.ShapeDtypeStruct((n, 128), jnp.float32),
    )(zp, denp, s)


def _tc_sage(sp, cntp, h, wl, wr, bf, n):
    grid = n // _ROWS

    def body(s_ref, c_ref, h_ref, wl_ref, wr_ref, b_ref, o_ref):
        cnt = jnp.maximum(jnp.sum(c_ref[...], axis=0), 1.0)
        mean = (s_ref[0] + s_ref[1]) / cnt[:, None]
        out = (jnp.dot(mean, wl_ref[...], preferred_element_type=jnp.float32)
               + jnp.dot(h_ref[...], wr_ref[...],
                         preferred_element_type=jnp.float32)
               + b_ref[...])
        o_ref[...] = jnp.maximum(out, 0.0)

    return pl.pallas_call(
        body,
        grid=(grid,),
        in_specs=[
            pl.BlockSpec((NC, _ROWS, 128), lambda i: (0, i, 0)),
            pl.BlockSpec((NW, _ROWS), lambda i: (0, i)),
            pl.BlockSpec((_ROWS, 128), lambda i: (i, 0)),
            pl.BlockSpec((128, 128), lambda i: (0, 0)),
            pl.BlockSpec((128, 128), lambda i: (0, 0)),
            pl.BlockSpec((1, 128), lambda i: (0, 0)),
        ],
        out_specs=pl.BlockSpec((_ROWS, 128), lambda i: (i, 0)),
        out_shape=jax.ShapeDtypeStruct((n, 128), jnp.float32),
    )(sp, cntp, h, wl, wr, bf)


def kernel(x, edge_index, params):
    n, d = x.shape
    e = edge_index.shape[1]
    assert d == 128

    src = edge_index[0].astype(jnp.int32)
    dst = edge_index[1].astype(jnp.int32)

    wc = jnp.concatenate([params["Wq"].T, params["Wk"].T,
                          params["Wv"].T, params["Wskip"].T], axis=1)
    bc = jnp.concatenate([params["bq"], params["bk"],
                          params["bv"], params["bskip"]])[None, :]
    a = jax.nn.sigmoid(params["res_alpha"])
    eye = jnp.eye(128, dtype=jnp.float32)
    wls, wrs, bfs = [], [], []
    for i in range(3):
        sbn = params["bn_g%d" % i] * lax.rsqrt(params["bn_v%d" % i] + 1e-5)
        wls.append(a * (params["W_l%d" % i].T * sbn[None, :]))
        wrs.append(a * (params["W_r%d" % i].T * sbn[None, :]) + (1.0 - a) * eye)
        bfs.append((a * ((params["b_l%d" % i] - params["bn_m%d" % i]) * sbn
                         + params["bn_b%d" % i]))[None, :])

    zeros_nd = jnp.zeros((n, 128), jnp.float32)

    qkvs = _tc_qkvs(x, wc, bc, n)
    q = qkvs[:, 0:128]
    k = qkvs[:, 128:256]
    v = qkvs[:, 256:384]
    s = qkvs[:, 384:512]

    zp, denp, cntp = _sc_attn(q, k, v, src, dst, zeros_nd, n, e)
    h = _tc_h(zp, denp, s, n)
    for i in range(3):
        sp = _sc_sage(h, src, dst, zeros_nd, n, e)
        h = _tc_sage(sp, cntp, h, wls[i], wrs[i], bfs[i], n)
    return h


# speedup vs baseline: 2.6748x; 2.6748x over previous
import functools

import jax
import jax.numpy as jnp
from jax import lax
from jax.experimental import pallas as pl
from jax.experimental.pallas import tpu as pltpu
from jax.experimental.pallas import tpu_sc as plsc

NC = 2
NS = 16
NW = NC * NS
LANES = 16
CHUNK = 80

_INV_SCALE = 1.0 / (128.0 ** 0.5)


def _mesh():
    return plsc.VectorSubcoreMesh(
        core_axis_name="c", subcore_axis_name="s", num_cores=NC, num_subcores=NS)


def _wid():
    return lax.axis_index("s") * NC + lax.axis_index("c")


def _sc_attn(q, k, v, src, dst, zeros_nd, zeros_n, n, e):
    ep = e // NW
    nch = ep // CHUNK
    assert ep % CHUNK == 0 and n % NS == 0
    rps = (n // NS) & ~7
    rextra = n - NS * rps
    assert rextra % 8 == 0

    def body(q_hbm, k_hbm, v_hbm, src_hbm, dst_hbm, zf_hbm, zn_hbm,
             z_out, den_out, cnt_out,
             srcv, dstv, qbuf, kbuf, vbuf, wbuf, onesbuf,
             z_sp, den_sp, cnt_sp):
        cid = lax.axis_index("c")
        sid = lax.axis_index("s")
        wid = _wid()

        nblk = n // CHUNK
        tpb = (nblk + NS - 1) // NS

        pltpu.sync_copy(zf_hbm.at[pl.ds(sid * rps, rps)],
                        z_sp.at[pl.ds(sid * rps, rps)])
        if rextra:
            @pl.when(sid == NS - 1)
            def _():
                pltpu.sync_copy(zf_hbm.at[pl.ds(NS * rps, rextra)],
                                z_sp.at[pl.ds(NS * rps, rextra)])
        pltpu.sync_copy(zn_hbm.at[pl.ds(0, CHUNK)], wbuf)

        def zblk(t, c):
            idx = sid + NS * t
            @pl.when(idx < nblk)
            def _():
                pltpu.sync_copy(wbuf, den_sp.at[pl.ds(idx * CHUNK, CHUNK)])
                pltpu.sync_copy(wbuf, cnt_sp.at[pl.ds(idx * CHUNK, CHUNK)])
            return c
        lax.fori_loop(0, tpb, zblk, 0)
        for g in range(CHUNK // LANES):
            onesbuf[pl.ds(g * LANES, LANES)] = jnp.ones((LANES,), jnp.float32)
        plsc.subcore_barrier()

        def chunk_body(ci, carry):
            base = wid * ep + ci * CHUNK
            pltpu.sync_copy(src_hbm.at[pl.ds(base, CHUNK)], srcv)
            pltpu.sync_copy(dst_hbm.at[pl.ds(base, CHUNK)], dstv)
            pltpu.sync_copy(q_hbm.at[dstv], qbuf)
            pltpu.sync_copy(k_hbm.at[srcv], kbuf)
            pltpu.sync_copy(v_hbm.at[srcv], vbuf)
            for g in range(CHUNK // LANES):
                rows = g * LANES + lax.iota(jnp.int32, LANES)
                dstg = dstv[pl.ds(g * LANES, LANES)]

                def dot_body(dd, acc):
                    for u in range(4):
                        col = jnp.full((LANES,), dd * 4 + u, jnp.int32)
                        qv = plsc.load_gather(qbuf, [rows, col])
                        kv = plsc.load_gather(kbuf, [rows, col])
                        acc = acc + qv * kv
                    return acc
                acc = lax.fori_loop(0, 32, dot_body,
                                    jnp.zeros((LANES,), jnp.float32))
                w = jnp.exp(acc * _INV_SCALE)
                wbuf[pl.ds(g * LANES, LANES)] = w

                def scale_body(dd, c):
                    for u in range(4):
                        col = jnp.full((LANES,), dd * 4 + u, jnp.int32)
                        vv = plsc.load_gather(vbuf, [rows, col])
                        plsc.store_scatter(vbuf, [rows, col], vv * w)
                    return c
                lax.fori_loop(0, 32, scale_body, 0)
            pltpu.sync_copy(vbuf, z_sp.at[dstv], add=True)
            pltpu.sync_copy(wbuf, den_sp.at[dstv], add=True)
            pltpu.sync_copy(onesbuf, cnt_sp.at[dstv], add=True)
            return carry
        lax.fori_loop(0, nch, chunk_body, 0)

        plsc.subcore_barrier()
        pltpu.sync_copy(z_sp.at[pl.ds(sid * rps, rps)],
                        z_out.at[cid, pl.ds(sid * rps, rps)])
        if rextra:
            @pl.when(sid == NS - 1)
            def _():
                pltpu.sync_copy(z_sp.at[pl.ds(NS * rps, rextra)],
                                z_out.at[cid, pl.ds(NS * rps, rextra)])

        def dblk(t, c):
            idx = sid + NS * t
            @pl.when(idx < nblk)
            def _():
                pltpu.sync_copy(den_sp.at[pl.ds(idx * CHUNK, CHUNK)], wbuf)
                pltpu.sync_copy(wbuf,
                                den_out.at[pl.ds(cid * n + idx * CHUNK, CHUNK)])
                pltpu.sync_copy(cnt_sp.at[pl.ds(idx * CHUNK, CHUNK)], onesbuf)
                pltpu.sync_copy(onesbuf,
                                cnt_out.at[pl.ds(cid * n + idx * CHUNK, CHUNK)])
            return c
        lax.fori_loop(0, tpb, dblk, 0)

    fn = pl.kernel(
        body,
        out_type=(jax.ShapeDtypeStruct((NC, n, 128), jnp.float32),
                  jax.ShapeDtypeStruct((NC * n,), jnp.float32),
                  jax.ShapeDtypeStruct((NC * n,), jnp.float32)),
        mesh=_mesh(),
        compiler_params=pltpu.CompilerParams(needs_layout_passes=False),
        scratch_types=[
            pltpu.VMEM((CHUNK,), jnp.int32),
            pltpu.VMEM((CHUNK,), jnp.int32),
            pltpu.VMEM((CHUNK, 128), jnp.float32),
            pltpu.VMEM((CHUNK, 128), jnp.float32),
            pltpu.VMEM((CHUNK, 128), jnp.float32),
            pltpu.VMEM((CHUNK,), jnp.float32),
            pltpu.VMEM((CHUNK,), jnp.float32),
            pltpu.VMEM_SHARED((n, 128), jnp.float32),
            pltpu.VMEM_SHARED((n,), jnp.float32),
            pltpu.VMEM_SHARED((n,), jnp.float32),
        ],
    )
    return fn(q, k, v, src, dst, zeros_nd, zeros_n)


def _sc_sage(h, src, dst, zeros_nd, n, e):
    ep = e // NW
    nch = ep // CHUNK
    rps = (n // NS) & ~7
    rextra = n - NS * rps

    def body(h_hbm, src_hbm, dst_hbm, zf_hbm, s_out,
             srcv, dstv, buf, s_sp):
        cid = lax.axis_index("c")
        sid = lax.axis_index("s")
        wid = _wid()
        pltpu.sync_copy(zf_hbm.at[pl.ds(sid * rps, rps)],
                        s_sp.at[pl.ds(sid * rps, rps)])
        if rextra:
            @pl.when(sid == NS - 1)
            def _():
                pltpu.sync_copy(zf_hbm.at[pl.ds(NS * rps, rextra)],
                                s_sp.at[pl.ds(NS * rps, rextra)])
        plsc.subcore_barrier()

        def chunk_body(ci, carry):
            base = wid * ep + ci * CHUNK
            pltpu.sync_copy(src_hbm.at[pl.ds(base, CHUNK)], srcv)
            pltpu.sync_copy(dst_hbm.at[pl.ds(base, CHUNK)], dstv)
            pltpu.sync_copy(h_hbm.at[srcv], buf)
            pltpu.sync_copy(buf, s_sp.at[dstv], add=True)
            return carry
        lax.fori_loop(0, nch, chunk_body, 0)

        plsc.subcore_barrier()
        pltpu.sync_copy(s_sp.at[pl.ds(sid * rps, rps)],
                        s_out.at[cid, pl.ds(sid * rps, rps)])
        if rextra:
            @pl.when(sid == NS - 1)
            def _():
                pltpu.sync_copy(s_sp.at[pl.ds(NS * rps, rextra)],
                                s_out.at[cid, pl.ds(NS * rps, rextra)])

    fn = pl.kernel(
        body,
        out_type=jax.ShapeDtypeStruct((NC, n, 128), jnp.float32),
        mesh=_mesh(),
        compiler_params=pltpu.CompilerParams(needs_layout_passes=False),
        scratch_types=[
            pltpu.VMEM((CHUNK,), jnp.int32),
            pltpu.VMEM((CHUNK,), jnp.int32),
            pltpu.VMEM((CHUNK, 128), jnp.float32),
            pltpu.VMEM_SHARED((n, 128), jnp.float32),
        ],
    )
    return fn(h, src, dst, zeros_nd)


_ROWS = 2000


def _tc_qkvs(x, wc, bc, n):
    grid = n // _ROWS

    def body(x_ref, w_ref, b_ref, o_ref):
        o_ref[...] = jnp.dot(x_ref[...], w_ref[...],
                             preferred_element_type=jnp.float32) + b_ref[...]

    return pl.pallas_call(
        body,
        grid=(grid,),
        in_specs=[
            pl.BlockSpec((_ROWS, 128), lambda i: (i, 0)),
            pl.BlockSpec((128, 512), lambda i: (0, 0)),
            pl.BlockSpec((1, 512), lambda i: (0, 0)),
        ],
        out_specs=pl.BlockSpec((_ROWS, 512), lambda i: (i, 0)),
        out_shape=jax.ShapeDtypeStruct((n, 512), jnp.float32),
    )(x, wc, bc)


def _tc_h(zp, denp, s, n):
    grid = n // _ROWS

    def body(z_ref, d_ref, s_ref, o_ref):
        z = z_ref[0] + z_ref[1]
        dsl = d_ref[0]
        den = dsl[0] + dsl[1] + 1e-16
        o_ref[...] = jnp.maximum(z / den[:, None] + s_ref[...], 0.0)

    return pl.pallas_call(
        body,
        grid=(grid,),
        in_specs=[
            pl.BlockSpec((NC, _ROWS, 128), lambda i: (0, i, 0)),
            pl.BlockSpec((1, NC, _ROWS), lambda i: (i, 0, 0)),
            pl.BlockSpec((_ROWS, 128), lambda i: (i, 0)),
        ],
        out_specs=pl.BlockSpec((_ROWS, 128), lambda i: (i, 0)),
        out_shape=jax.ShapeDtypeStruct((n, 128), jnp.float32),
    )(zp, denp, s)


def _tc_sage(sp, cntp, h, wl, wr, bf, n):
    grid = n // _ROWS

    def body(s_ref, c_ref, h_ref, wl_ref, wr_ref, b_ref, o_ref):
        csl = c_ref[0]
        cnt = jnp.maximum(csl[0] + csl[1], 1.0)
        mean = (s_ref[0] + s_ref[1]) / cnt[:, None]
        out = (jnp.dot(mean, wl_ref[...], preferred_element_type=jnp.float32)
               + jnp.dot(h_ref[...], wr_ref[...],
                         preferred_element_type=jnp.float32)
               + b_ref[...])
        o_ref[...] = jnp.maximum(out, 0.0)

    return pl.pallas_call(
        body,
        grid=(grid,),
        in_specs=[
            pl.BlockSpec((NC, _ROWS, 128), lambda i: (0, i, 0)),
            pl.BlockSpec((1, NC, _ROWS), lambda i: (i, 0, 0)),
            pl.BlockSpec((_ROWS, 128), lambda i: (i, 0)),
            pl.BlockSpec((128, 128), lambda i: (0, 0)),
            pl.BlockSpec((128, 128), lambda i: (0, 0)),
            pl.BlockSpec((1, 128), lambda i: (0, 0)),
        ],
        out_specs=pl.BlockSpec((_ROWS, 128), lambda i: (i, 0)),
        out_shape=jax.ShapeDtypeStruct((n, 128), jnp.float32),
    )(sp, cntp, h, wl, wr, bf)


def kernel(x, edge_index, params):
    n, d = x.shape
    e = edge_index.shape[1]
    assert d == 128

    src = edge_index[0].astype(jnp.int32)
    dst = edge_index[1].astype(jnp.int32)

    wc = jnp.concatenate([params["Wq"].T, params["Wk"].T,
                          params["Wv"].T, params["Wskip"].T], axis=1)
    bc = jnp.concatenate([params["bq"], params["bk"],
                          params["bv"], params["bskip"]])[None, :]
    a = jax.nn.sigmoid(params["res_alpha"])
    eye = jnp.eye(128, dtype=jnp.float32)
    wls, wrs, bfs = [], [], []
    for i in range(3):
        sbn = params["bn_g%d" % i] * lax.rsqrt(params["bn_v%d" % i] + 1e-5)
        wls.append(a * (params["W_l%d" % i].T * sbn[None, :]))
        wrs.append(a * (params["W_r%d" % i].T * sbn[None, :]) + (1.0 - a) * eye)
        bfs.append((a * ((params["b_l%d" % i] - params["bn_m%d" % i]) * sbn
                         + params["bn_b%d" % i]))[None, :])

    zeros_nd = jnp.zeros((n, 128), jnp.float32)
    zeros_n = jnp.zeros((n,), jnp.float32)

    qkvs = _tc_qkvs(x, wc, bc, n)
    q = qkvs[:, 0:128]
    k = qkvs[:, 128:256]
    v = qkvs[:, 256:384]
    s = qkvs[:, 384:512]

    zp, denp, cntp = _sc_attn(q, k, v, src, dst, zeros_nd, zeros_n, n, e)
    grid = n // _ROWS
    denp = denp.reshape(NC, grid, _ROWS).transpose(1, 0, 2)
    cntp = cntp.reshape(NC, grid, _ROWS).transpose(1, 0, 2)
    h = _tc_h(zp, denp, s, n)
    for i in range(3):
        sp = _sc_sage(h, src, dst, zeros_nd, n, e)
        h = _tc_sage(sp, cntp, h, wls[i], wrs[i], bfs[i], n)
    return h


# speedup vs baseline: 5.4989x; 2.0558x over previous
import functools

import jax
import jax.numpy as jnp
from jax import lax
from jax.experimental import pallas as pl
from jax.experimental.pallas import tpu as pltpu
from jax.experimental.pallas import tpu_sc as plsc

NC = 2
NS = 16
NW = NC * NS
LANES = 16
CHUNK = 80

_INV_SCALE = 1.0 / (128.0 ** 0.5)


def _mesh():
    return plsc.VectorSubcoreMesh(
        core_axis_name="c", subcore_axis_name="s", num_cores=NC, num_subcores=NS)


def _wid():
    return lax.axis_index("s") * NC + lax.axis_index("c")


def _sc_attn(q, k, v, src, dst, zeros_nd, zeros_n, n, e):
    ep = e // NW
    nch = ep // CHUNK
    assert ep % CHUNK == 0 and n % NS == 0
    rps = (n // NS) & ~7
    rextra = n - NS * rps
    assert rextra % 8 == 0

    def body(q_hbm, k_hbm, v_hbm, src_hbm, dst_hbm, zf_hbm, zn_hbm,
             z_out, den_out, cnt_out,
             srcv, dstv, qbuf, kbuf, vbuf, wbuf, onesbuf,
             z_sp, den_sp, cnt_sp):
        cid = lax.axis_index("c")
        sid = lax.axis_index("s")
        wid = _wid()

        nblk = n // CHUNK
        tpb = (nblk + NS - 1) // NS

        pltpu.sync_copy(zf_hbm.at[pl.ds(sid * rps, rps)],
                        z_sp.at[pl.ds(sid * rps, rps)])
        if rextra:
            @pl.when(sid == NS - 1)
            def _():
                pltpu.sync_copy(zf_hbm.at[pl.ds(NS * rps, rextra)],
                                z_sp.at[pl.ds(NS * rps, rextra)])
        pltpu.sync_copy(zn_hbm.at[pl.ds(0, CHUNK)], wbuf)

        def zblk(t, c):
            idx = sid + NS * t
            @pl.when(idx < nblk)
            def _():
                pltpu.sync_copy(wbuf, den_sp.at[pl.ds(idx * CHUNK, CHUNK)])
                pltpu.sync_copy(wbuf, cnt_sp.at[pl.ds(idx * CHUNK, CHUNK)])
            return c
        lax.fori_loop(0, tpb, zblk, 0)
        for g in range(CHUNK // LANES):
            onesbuf[pl.ds(g * LANES, LANES)] = jnp.ones((LANES,), jnp.float32)
        plsc.subcore_barrier()

        def chunk_body(ci, carry):
            base = wid * ep + ci * CHUNK
            pltpu.sync_copy(src_hbm.at[pl.ds(base, CHUNK)], srcv)
            pltpu.sync_copy(dst_hbm.at[pl.ds(base, CHUNK)], dstv)
            pltpu.sync_copy(q_hbm.at[dstv], qbuf)
            pltpu.sync_copy(k_hbm.at[srcv], kbuf)
            pltpu.sync_copy(v_hbm.at[srcv], vbuf)
            lanes = lax.iota(jnp.int32, LANES)
            for g in range(CHUNK // LANES):
                rows = g * LANES + lanes
                dstg = dstv[pl.ds(g * LANES, LANES)]

                def dot_body(dd, acc):
                    for u in range(4):
                        col = (lanes + (dd * 4 + u)) & 127
                        qv = plsc.load_gather(qbuf, [rows, col])
                        kv = plsc.load_gather(kbuf, [rows, col])
                        acc = acc + qv * kv
                    return acc
                acc = lax.fori_loop(0, 32, dot_body,
                                    jnp.zeros((LANES,), jnp.float32))
                w = jnp.exp(acc * _INV_SCALE)
                wbuf[pl.ds(g * LANES, LANES)] = w

                def scale_body(dd, c):
                    for u in range(4):
                        col = (lanes + (dd * 4 + u)) & 127
                        vv = plsc.load_gather(vbuf, [rows, col])
                        plsc.store_scatter(vbuf, [rows, col], vv * w)
                    return c
                lax.fori_loop(0, 32, scale_body, 0)
            pltpu.sync_copy(vbuf, z_sp.at[dstv], add=True)
            pltpu.sync_copy(wbuf, den_sp.at[dstv], add=True)
            pltpu.sync_copy(onesbuf, cnt_sp.at[dstv], add=True)
            return carry
        lax.fori_loop(0, nch, chunk_body, 0)

        plsc.subcore_barrier()
        pltpu.sync_copy(z_sp.at[pl.ds(sid * rps, rps)],
                        z_out.at[cid, pl.ds(sid * rps, rps)])
        if rextra:
            @pl.when(sid == NS - 1)
            def _():
                pltpu.sync_copy(z_sp.at[pl.ds(NS * rps, rextra)],
                                z_out.at[cid, pl.ds(NS * rps, rextra)])

        def dblk(t, c):
            idx = sid + NS * t
            @pl.when(idx < nblk)
            def _():
                pltpu.sync_copy(den_sp.at[pl.ds(idx * CHUNK, CHUNK)], wbuf)
                pltpu.sync_copy(wbuf,
                                den_out.at[pl.ds(cid * n + idx * CHUNK, CHUNK)])
                pltpu.sync_copy(cnt_sp.at[pl.ds(idx * CHUNK, CHUNK)], onesbuf)
                pltpu.sync_copy(onesbuf,
                                cnt_out.at[pl.ds(cid * n + idx * CHUNK, CHUNK)])
            return c
        lax.fori_loop(0, tpb, dblk, 0)

    fn = pl.kernel(
        body,
        out_type=(jax.ShapeDtypeStruct((NC, n, 128), jnp.float32),
                  jax.ShapeDtypeStruct((NC * n,), jnp.float32),
                  jax.ShapeDtypeStruct((NC * n,), jnp.float32)),
        mesh=_mesh(),
        compiler_params=pltpu.CompilerParams(needs_layout_passes=False),
        scratch_types=[
            pltpu.VMEM((CHUNK,), jnp.int32),
            pltpu.VMEM((CHUNK,), jnp.int32),
            pltpu.VMEM((CHUNK, 128), jnp.float32),
            pltpu.VMEM((CHUNK, 128), jnp.float32),
            pltpu.VMEM((CHUNK, 128), jnp.float32),
            pltpu.VMEM((CHUNK,), jnp.float32),
            pltpu.VMEM((CHUNK,), jnp.float32),
            pltpu.VMEM_SHARED((n, 128), jnp.float32),
            pltpu.VMEM_SHARED((n,), jnp.float32),
            pltpu.VMEM_SHARED((n,), jnp.float32),
        ],
    )
    return fn(q, k, v, src, dst, zeros_nd, zeros_n)


def _sc_sage(h, src, dst, zeros_nd, n, e):
    ep = e // NW
    nch = ep // CHUNK
    rps = (n // NS) & ~7
    rextra = n - NS * rps

    def body(h_hbm, src_hbm, dst_hbm, zf_hbm, s_out,
             srcv, dstv, buf, s_sp):
        cid = lax.axis_index("c")
        sid = lax.axis_index("s")
        wid = _wid()
        pltpu.sync_copy(zf_hbm.at[pl.ds(sid * rps, rps)],
                        s_sp.at[pl.ds(sid * rps, rps)])
        if rextra:
            @pl.when(sid == NS - 1)
            def _():
                pltpu.sync_copy(zf_hbm.at[pl.ds(NS * rps, rextra)],
                                s_sp.at[pl.ds(NS * rps, rextra)])
        plsc.subcore_barrier()

        def chunk_body(ci, carry):
            base = wid * ep + ci * CHUNK
            pltpu.sync_copy(src_hbm.at[pl.ds(base, CHUNK)], srcv)
            pltpu.sync_copy(dst_hbm.at[pl.ds(base, CHUNK)], dstv)
            pltpu.sync_copy(h_hbm.at[srcv], buf)
            pltpu.sync_copy(buf, s_sp.at[dstv], add=True)
            return carry
        lax.fori_loop(0, nch, chunk_body, 0)

        plsc.subcore_barrier()
        pltpu.sync_copy(s_sp.at[pl.ds(sid * rps, rps)],
                        s_out.at[cid, pl.ds(sid * rps, rps)])
        if rextra:
            @pl.when(sid == NS - 1)
            def _():
                pltpu.sync_copy(s_sp.at[pl.ds(NS * rps, rextra)],
                                s_out.at[cid, pl.ds(NS * rps, rextra)])

    fn = pl.kernel(
        body,
        out_type=jax.ShapeDtypeStruct((NC, n, 128), jnp.float32),
        mesh=_mesh(),
        compiler_params=pltpu.CompilerParams(needs_layout_passes=False),
        scratch_types=[
            pltpu.VMEM((CHUNK,), jnp.int32),
            pltpu.VMEM((CHUNK,), jnp.int32),
            pltpu.VMEM((CHUNK, 128), jnp.float32),
            pltpu.VMEM_SHARED((n, 128), jnp.float32),
        ],
    )
    return fn(h, src, dst, zeros_nd)


_ROWS = 2000


def _tc_qkvs(x, wc, bc, n):
    grid = n // _ROWS

    def body(x_ref, w_ref, b_ref, o_ref):
        o_ref[...] = jnp.dot(x_ref[...], w_ref[...],
                             preferred_element_type=jnp.float32) + b_ref[...]

    return pl.pallas_call(
        body,
        grid=(grid,),
        in_specs=[
            pl.BlockSpec((_ROWS, 128), lambda i: (i, 0)),
            pl.BlockSpec((128, 512), lambda i: (0, 0)),
            pl.BlockSpec((1, 512), lambda i: (0, 0)),
        ],
        out_specs=pl.BlockSpec((_ROWS, 512), lambda i: (i, 0)),
        out_shape=jax.ShapeDtypeStruct((n, 512), jnp.float32),
    )(x, wc, bc)


def _tc_h(zp, denp, s, n):
    grid = n // _ROWS

    def body(z_ref, d_ref, s_ref, o_ref):
        z = z_ref[0] + z_ref[1]
        dsl = d_ref[0]
        den = dsl[0] + dsl[1] + 1e-16
        o_ref[...] = jnp.maximum(z / den[:, None] + s_ref[...], 0.0)

    return pl.pallas_call(
        body,
        grid=(grid,),
        in_specs=[
            pl.BlockSpec((NC, _ROWS, 128), lambda i: (0, i, 0)),
            pl.BlockSpec((1, NC, _ROWS), lambda i: (i, 0, 0)),
            pl.BlockSpec((_ROWS, 128), lambda i: (i, 0)),
        ],
        out_specs=pl.BlockSpec((_ROWS, 128), lambda i: (i, 0)),
        out_shape=jax.ShapeDtypeStruct((n, 128), jnp.float32),
    )(zp, denp, s)


def _tc_sage(sp, cntp, h, wl, wr, bf, n):
    grid = n // _ROWS

    def body(s_ref, c_ref, h_ref, wl_ref, wr_ref, b_ref, o_ref):
        csl = c_ref[0]
        cnt = jnp.maximum(csl[0] + csl[1], 1.0)
        mean = (s_ref[0] + s_ref[1]) / cnt[:, None]
        out = (jnp.dot(mean, wl_ref[...], preferred_element_type=jnp.float32)
               + jnp.dot(h_ref[...], wr_ref[...],
                         preferred_element_type=jnp.float32)
               + b_ref[...])
        o_ref[...] = jnp.maximum(out, 0.0)

    return pl.pallas_call(
        body,
        grid=(grid,),
        in_specs=[
            pl.BlockSpec((NC, _ROWS, 128), lambda i: (0, i, 0)),
            pl.BlockSpec((1, NC, _ROWS), lambda i: (i, 0, 0)),
            pl.BlockSpec((_ROWS, 128), lambda i: (i, 0)),
            pl.BlockSpec((128, 128), lambda i: (0, 0)),
            pl.BlockSpec((128, 128), lambda i: (0, 0)),
            pl.BlockSpec((1, 128), lambda i: (0, 0)),
        ],
        out_specs=pl.BlockSpec((_ROWS, 128), lambda i: (i, 0)),
        out_shape=jax.ShapeDtypeStruct((n, 128), jnp.float32),
    )(sp, cntp, h, wl, wr, bf)


def kernel(x, edge_index, params):
    n, d = x.shape
    e = edge_index.shape[1]
    assert d == 128

    src = edge_index[0].astype(jnp.int32)
    dst = edge_index[1].astype(jnp.int32)

    wc = jnp.concatenate([params["Wq"].T, params["Wk"].T,
                          params["Wv"].T, params["Wskip"].T], axis=1)
    bc = jnp.concatenate([params["bq"], params["bk"],
                          params["bv"], params["bskip"]])[None, :]
    a = jax.nn.sigmoid(params["res_alpha"])
    eye = jnp.eye(128, dtype=jnp.float32)
    wls, wrs, bfs = [], [], []
    for i in range(3):
        sbn = params["bn_g%d" % i] * lax.rsqrt(params["bn_v%d" % i] + 1e-5)
        wls.append(a * (params["W_l%d" % i].T * sbn[None, :]))
        wrs.append(a * (params["W_r%d" % i].T * sbn[None, :]) + (1.0 - a) * eye)
        bfs.append((a * ((params["b_l%d" % i] - params["bn_m%d" % i]) * sbn
                         + params["bn_b%d" % i]))[None, :])

    zeros_nd = jnp.zeros((n, 128), jnp.float32)
    zeros_n = jnp.zeros((n,), jnp.float32)

    qkvs = _tc_qkvs(x, wc, bc, n)
    q = qkvs[:, 0:128]
    k = qkvs[:, 128:256]
    v = qkvs[:, 256:384]
    s = qkvs[:, 384:512]

    zp, denp, cntp = _sc_attn(q, k, v, src, dst, zeros_nd, zeros_n, n, e)
    grid = n // _ROWS
    denp = denp.reshape(NC, grid, _ROWS).transpose(1, 0, 2)
    cntp = cntp.reshape(NC, grid, _ROWS).transpose(1, 0, 2)
    h = _tc_h(zp, denp, s, n)
    for i in range(3):
        sp = _sc_sage(h, src, dst, zeros_nd, n, e)
        h = _tc_sage(sp, cntp, h, wls[i], wrs[i], bfs[i], n)
    return h


# speedup vs baseline: 8.1658x; 1.4850x over previous
import jax
import jax.numpy as jnp
from jax import lax
from jax.experimental import pallas as pl
from jax.experimental.pallas import tpu as pltpu
from jax.experimental.pallas import tpu_sc as plsc

NC = 2
NS = 16
NW = NC * NS
LANES = 16
CHUNK = 80

_INV_SCALE = 1.0 / (128.0 ** 0.5)


def _mesh():
    return plsc.VectorSubcoreMesh(
        core_axis_name="c", subcore_axis_name="s", num_cores=NC, num_subcores=NS)


def _wid():
    return lax.axis_index("s") * NC + lax.axis_index("c")


def _zero_nd(zf_hbm, acc_sp, sid, rps, rextra):
    pltpu.sync_copy(zf_hbm.at[pl.ds(sid * rps, rps)],
                    acc_sp.at[pl.ds(sid * rps, rps)])
    if rextra:
        @pl.when(sid == NS - 1)
        def _():
            pltpu.sync_copy(zf_hbm.at[pl.ds(NS * rps, rextra)],
                            acc_sp.at[pl.ds(NS * rps, rextra)])


def _dump_nd(acc_sp, out, cid, sid, rps, rextra):
    pltpu.sync_copy(acc_sp.at[pl.ds(sid * rps, rps)],
                    out.at[cid, pl.ds(sid * rps, rps)])
    if rextra:
        @pl.when(sid == NS - 1)
        def _():
            pltpu.sync_copy(acc_sp.at[pl.ds(NS * rps, rextra)],
                            out.at[cid, pl.ds(NS * rps, rextra)])


def _sc_attn_w(q, k, src, dst, zeros_n, n, e):
    ep = e // NW
    nch = ep // CHUNK
    assert ep % CHUNK == 0 and CHUNK % LANES == 0

    def body(q_hbm, k_hbm, src_hbm, dst_hbm, zn_hbm,
             w_out, den_out, cnt_out, *scr):
        srcv = list(scr[0:4])
        dstv = list(scr[4:8])
        qb = list(scr[8:10])
        kb = list(scr[10:12])
        wb = list(scr[12:14])
        onesbuf = scr[14]
        den_sp, cnt_sp = scr[15], scr[16]
        semi = list(scr[17:21])
        semg = list(scr[21:23])

        cid = lax.axis_index("c")
        sid = lax.axis_index("s")
        wid = _wid()
        base0 = wid * ep

        nblk = n // CHUNK
        tpb = (nblk + NS - 1) // NS
        pltpu.sync_copy(zn_hbm.at[pl.ds(0, CHUNK)], wb[0])

        def zblk(t, c):
            idx = sid + NS * t
            @pl.when(idx < nblk)
            def _():
                pltpu.sync_copy(wb[0], den_sp.at[pl.ds(idx * CHUNK, CHUNK)])
                pltpu.sync_copy(wb[0], cnt_sp.at[pl.ds(idx * CHUNK, CHUNK)])
            return c
        lax.fori_loop(0, tpb, zblk, 0)
        for g in range(CHUNK // LANES):
            onesbuf[pl.ds(g * LANES, LANES)] = jnp.ones((LANES,), jnp.float32)
        plsc.subcore_barrier()

        pltpu.async_copy(src_hbm.at[pl.ds(base0, CHUNK)], srcv[0], semi[0])
        pltpu.async_copy(dst_hbm.at[pl.ds(base0, CHUNK)], dstv[0], semi[0])
        pltpu.async_copy(src_hbm.at[pl.ds(base0 + CHUNK, CHUNK)], srcv[1],
                         semi[1])
        pltpu.async_copy(dst_hbm.at[pl.ds(base0 + CHUNK, CHUNK)], dstv[1],
                         semi[1])
        pltpu.make_async_copy(src_hbm.at[pl.ds(base0, CHUNK)], srcv[0],
                              semi[0]).wait()
        pltpu.make_async_copy(dst_hbm.at[pl.ds(base0, CHUNK)], dstv[0],
                              semi[0]).wait()
        pltpu.async_copy(q_hbm.at[dstv[0]], qb[0], semg[0])
        pltpu.async_copy(k_hbm.at[srcv[0]], kb[0], semg[0])

        lanes = lax.iota(jnp.int32, LANES)

        def slot(ci, u):
            b = u % 2
            u1, b1 = (u + 1) % 4, (u + 1) % 2
            u2 = (u + 2) % 4
            live = ci < nch
            base = base0 + ci * CHUNK

            @pl.when(live)
            def _():
                pltpu.make_async_copy(q_hbm.at[dstv[u]], qb[b],
                                      semg[b]).wait()
                pltpu.make_async_copy(k_hbm.at[srcv[u]], kb[b],
                                      semg[b]).wait()
                for g in range(CHUNK // LANES):
                    rows = g * LANES + lanes

                    def dot_body(dd, acc):
                        for uu in range(4):
                            col = (lanes + (dd * 4 + uu)) & 127
                            qv = plsc.load_gather(qb[b], [rows, col])
                            kv = plsc.load_gather(kb[b], [rows, col])
                            acc = acc + qv * kv
                        return acc
                    acc = lax.fori_loop(0, 32, dot_body,
                                        jnp.zeros((LANES,), jnp.float32))
                    wb[b][pl.ds(g * LANES, LANES)] = (
                        jnp.exp(acc * _INV_SCALE))

            @pl.when(ci + 1 < nch)
            def _():
                base1 = base0 + (ci + 1) * CHUNK
                pltpu.make_async_copy(src_hbm.at[pl.ds(base1, CHUNK)],
                                      srcv[u1], semi[u1]).wait()
                pltpu.make_async_copy(dst_hbm.at[pl.ds(base1, CHUNK)],
                                      dstv[u1], semi[u1]).wait()
                pltpu.async_copy(q_hbm.at[dstv[u1]], qb[b1], semg[b1])
                pltpu.async_copy(k_hbm.at[srcv[u1]], kb[b1], semg[b1])

            @pl.when(live)
            def _():
                pltpu.sync_copy(wb[b], w_out.at[pl.ds(base, CHUNK)])
                pltpu.sync_copy(wb[b], den_sp.at[dstv[u]], add=True)
                pltpu.sync_copy(onesbuf, cnt_sp.at[dstv[u]], add=True)

            @pl.when(ci + 2 < nch)
            def _():
                base2 = base0 + (ci + 2) * CHUNK
                pltpu.async_copy(src_hbm.at[pl.ds(base2, CHUNK)], srcv[u2],
                                 semi[u2])
                pltpu.async_copy(dst_hbm.at[pl.ds(base2, CHUNK)], dstv[u2],
                                 semi[u2])

        def group(gi, c):
            for u in range(4):
                slot(gi * 4 + u, u)
            return c
        lax.fori_loop(0, (nch + 3) // 4, group, 0)

        plsc.subcore_barrier()

        def dblk(t, c):
            idx = sid + NS * t
            @pl.when(idx < nblk)
            def _():
                pltpu.sync_copy(den_sp.at[pl.ds(idx * CHUNK, CHUNK)], wb[0])
                pltpu.sync_copy(
                    wb[0], den_out.at[pl.ds(cid * n + idx * CHUNK, CHUNK)])
                pltpu.sync_copy(cnt_sp.at[pl.ds(idx * CHUNK, CHUNK)], wb[1])
                pltpu.sync_copy(
                    wb[1], cnt_out.at[pl.ds(cid * n + idx * CHUNK, CHUNK)])
            return c
        lax.fori_loop(0, tpb, dblk, 0)

    fn = pl.kernel(
        body,
        out_type=(jax.ShapeDtypeStruct((e,), jnp.float32),
                  jax.ShapeDtypeStruct((NC * n,), jnp.float32),
                  jax.ShapeDtypeStruct((NC * n,), jnp.float32)),
        mesh=_mesh(),
        compiler_params=pltpu.CompilerParams(needs_layout_passes=False),
        scratch_types=(
            [pltpu.VMEM((CHUNK,), jnp.int32) for _ in range(8)]
            + [pltpu.VMEM((CHUNK, 128), jnp.float32) for _ in range(4)]
            + [pltpu.VMEM((CHUNK,), jnp.float32) for _ in range(3)]
            + [pltpu.VMEM_SHARED((n,), jnp.float32) for _ in range(2)]
            + [pltpu.SemaphoreType.DMA for _ in range(6)]
        ),
    )
    return fn(q, k, src, dst, zeros_n)


def _sc_scatter(h, src, dst, zeros_nd, n, e, w=None):
    ep = e // NW
    nch = ep // CHUNK
    rps = (n // NS) & ~7
    rextra = n - NS * rps
    scaled = w is not None

    def body(*args):
        if scaled:
            (h_hbm, src_hbm, dst_hbm, zf_hbm, w_hbm, s_out, *scr) = args
        else:
            (h_hbm, src_hbm, dst_hbm, zf_hbm, s_out, *scr) = args
        srcv = list(scr[0:4])
        dstv = list(scr[4:8])
        buf = list(scr[8:10])
        s_sp = scr[10]
        semi = list(scr[11:15])
        semg = list(scr[15:17])
        wv = list(scr[17:21]) if scaled else None

        cid = lax.axis_index("c")
        sid = lax.axis_index("s")
        wid = _wid()
        base0 = wid * ep

        _zero_nd(zf_hbm, s_sp, sid, rps, rextra)
        plsc.subcore_barrier()

        def fetch_idx(ci, u):
            base = base0 + ci * CHUNK
            pltpu.async_copy(src_hbm.at[pl.ds(base, CHUNK)], srcv[u], semi[u])
            pltpu.async_copy(dst_hbm.at[pl.ds(base, CHUNK)], dstv[u], semi[u])
            if scaled:
                pltpu.async_copy(w_hbm.at[pl.ds(base, CHUNK)], wv[u], semi[u])

        def wait_idx(ci, u):
            base = base0 + ci * CHUNK
            pltpu.make_async_copy(src_hbm.at[pl.ds(base, CHUNK)], srcv[u],
                                  semi[u]).wait()
            pltpu.make_async_copy(dst_hbm.at[pl.ds(base, CHUNK)], dstv[u],
                                  semi[u]).wait()
            if scaled:
                pltpu.make_async_copy(w_hbm.at[pl.ds(base, CHUNK)], wv[u],
                                      semi[u]).wait()

        fetch_idx(0, 0)
        fetch_idx(1, 1)
        wait_idx(0, 0)
        pltpu.async_copy(h_hbm.at[srcv[0]], buf[0], semg[0])

        lanes = lax.iota(jnp.int32, LANES)

        def slot(ci, u):
            b = u % 2
            u1, b1 = (u + 1) % 4, (u + 1) % 2
            u2 = (u + 2) % 4
            live = ci < nch

            @pl.when(live)
            def _():
                pltpu.make_async_copy(h_hbm.at[srcv[u]], buf[b],
                                      semg[b]).wait()
                if scaled:
                    for g in range(CHUNK // LANES):
                        rows = g * LANES + lanes
                        w16 = wv[u][pl.ds(g * LANES, LANES)]

                        def scale_body(dd, c):
                            for uu in range(4):
                                col = (lanes + (dd * 4 + uu)) & 127
                                vvv = plsc.load_gather(buf[b], [rows, col])
                                plsc.store_scatter(buf[b], [rows, col],
                                                   vvv * w16)
                            return c
                        lax.fori_loop(0, 32, scale_body, 0)

            @pl.when(ci + 1 < nch)
            def _():
                wait_idx(ci + 1, u1)
                pltpu.async_copy(h_hbm.at[srcv[u1]], buf[b1], semg[b1])

            @pl.when(live)
            def _():
                pltpu.sync_copy(buf[b], s_sp.at[dstv[u]], add=True)

            @pl.when(ci + 2 < nch)
            def _():
                fetch_idx(ci + 2, u2)

        def group(gi, c):
            for u in range(4):
                slot(gi * 4 + u, u)
            return c
        lax.fori_loop(0, (nch + 3) // 4, group, 0)

        plsc.subcore_barrier()
        _dump_nd(s_sp, s_out, cid, sid, rps, rextra)

    fn = pl.kernel(
        body,
        out_type=jax.ShapeDtypeStruct((NC, n, 128), jnp.float32),
        mesh=_mesh(),
        compiler_params=pltpu.CompilerParams(needs_layout_passes=False),
        scratch_types=(
            [pltpu.VMEM((CHUNK,), jnp.int32) for _ in range(8)]
            + [pltpu.VMEM((CHUNK, 128), jnp.float32) for _ in range(2)]
            + [pltpu.VMEM_SHARED((n, 128), jnp.float32)]
            + [pltpu.SemaphoreType.DMA for _ in range(6)]
            + ([pltpu.VMEM((CHUNK,), jnp.float32) for _ in range(4)]
               if scaled else [])
        ),
    )
    if scaled:
        return fn(h, src, dst, zeros_nd, w)
    return fn(h, src, dst, zeros_nd)


_ROWS = 2000


def _tc_qkvs(x, wc, bc, n):
    grid = n // _ROWS

    def body(x_ref, w_ref, b_ref, o_ref):
        o_ref[...] = jnp.dot(x_ref[...], w_ref[...],
                             preferred_element_type=jnp.float32) + b_ref[...]

    return pl.pallas_call(
        body,
        grid=(grid,),
        in_specs=[
            pl.BlockSpec((_ROWS, 128), lambda i: (i, 0)),
            pl.BlockSpec((128, 512), lambda i: (0, 0)),
            pl.BlockSpec((1, 512), lambda i: (0, 0)),
        ],
        out_specs=pl.BlockSpec((_ROWS, 512), lambda i: (i, 0)),
        out_shape=jax.ShapeDtypeStruct((n, 512), jnp.float32),
    )(x, wc, bc)


def _tc_h(zp, denp, s, n):
    grid = n // _ROWS

    def body(z_ref, d_ref, s_ref, o_ref):
        z = z_ref[0] + z_ref[1]
        dsl = d_ref[0]
        den = dsl[0] + dsl[1] + 1e-16
        o_ref[...] = jnp.maximum(z / den[:, None] + s_ref[...], 0.0)

    return pl.pallas_call(
        body,
        grid=(grid,),
        in_specs=[
            pl.BlockSpec((NC, _ROWS, 128), lambda i: (0, i, 0)),
            pl.BlockSpec((1, NC, _ROWS), lambda i: (i, 0, 0)),
            pl.BlockSpec((_ROWS, 128), lambda i: (i, 0)),
        ],
        out_specs=pl.BlockSpec((_ROWS, 128), lambda i: (i, 0)),
        out_shape=jax.ShapeDtypeStruct((n, 128), jnp.float32),
    )(zp, denp, s)


def _tc_sage(sp, cntp, h, wl, wr, bf, n):
    grid = n // _ROWS

    def body(s_ref, c_ref, h_ref, wl_ref, wr_ref, b_ref, o_ref):
        csl = c_ref[0]
        cnt = jnp.maximum(csl[0] + csl[1], 1.0)
        mean = (s_ref[0] + s_ref[1]) / cnt[:, None]
        out = (jnp.dot(mean, wl_ref[...], preferred_element_type=jnp.float32)
               + jnp.dot(h_ref[...], wr_ref[...],
                         preferred_element_type=jnp.float32)
               + b_ref[...])
        o_ref[...] = jnp.maximum(out, 0.0)

    return pl.pallas_call(
        body,
        grid=(grid,),
        in_specs=[
            pl.BlockSpec((NC, _ROWS, 128), lambda i: (0, i, 0)),
            pl.BlockSpec((1, NC, _ROWS), lambda i: (i, 0, 0)),
            pl.BlockSpec((_ROWS, 128), lambda i: (i, 0)),
            pl.BlockSpec((128, 128), lambda i: (0, 0)),
            pl.BlockSpec((128, 128), lambda i: (0, 0)),
            pl.BlockSpec((1, 128), lambda i: (0, 0)),
        ],
        out_specs=pl.BlockSpec((_ROWS, 128), lambda i: (i, 0)),
        out_shape=jax.ShapeDtypeStruct((n, 128), jnp.float32),
    )(sp, cntp, h, wl, wr, bf)


def kernel(x, edge_index, params):
    n, d = x.shape
    e = edge_index.shape[1]
    assert d == 128

    src = edge_index[0].astype(jnp.int32)
    dst = edge_index[1].astype(jnp.int32)

    wc = jnp.concatenate([params["Wq"].T, params["Wk"].T,
                          params["Wv"].T, params["Wskip"].T], axis=1)
    bc = jnp.concatenate([params["bq"], params["bk"],
                          params["bv"], params["bskip"]])[None, :]
    a = jax.nn.sigmoid(params["res_alpha"])
    eye = jnp.eye(128, dtype=jnp.float32)
    wls, wrs, bfs = [], [], []
    for i in range(3):
        sbn = params["bn_g%d" % i] * lax.rsqrt(params["bn_v%d" % i] + 1e-5)
        wls.append(a * (params["W_l%d" % i].T * sbn[None, :]))
        wrs.append(a * (params["W_r%d" % i].T * sbn[None, :]) + (1.0 - a) * eye)
        bfs.append((a * ((params["b_l%d" % i] - params["bn_m%d" % i]) * sbn
                         + params["bn_b%d" % i]))[None, :])

    zeros_nd = jnp.zeros((n, 128), jnp.float32)
    zeros_n = jnp.zeros((n,), jnp.float32)

    qkvs = _tc_qkvs(x, wc, bc, n)
    q = qkvs[:, 0:128]
    k = qkvs[:, 128:256]
    v = qkvs[:, 256:384]
    s = qkvs[:, 384:512]

    we, denp, cntp = _sc_attn_w(q, k, src, dst, zeros_n, n, e)
    zp = _sc_scatter(v, src, dst, zeros_nd, n, e, w=we)

    grid = n // _ROWS
    denp = denp.reshape(NC, grid, _ROWS).transpose(1, 0, 2)
    cntp = cntp.reshape(NC, grid, _ROWS).transpose(1, 0, 2)

    h = _tc_h(zp, denp, s, n)
    for i in range(3):
        sp = _sc_scatter(h, src, dst, zeros_nd, n, e)
        h = _tc_sage(sp, cntp, h, wls[i], wrs[i], bfs[i], n)
    return h


# speedup vs baseline: 8.8408x; 1.0827x over previous
import jax
import jax.numpy as jnp
from jax import lax
from jax.experimental import pallas as pl
from jax.experimental.pallas import tpu as pltpu
from jax.experimental.pallas import tpu_sc as plsc

NC = 2
NS = 16
NW = NC * NS
LANES = 16
CHUNK = 80

_INV_SCALE = 1.0 / (128.0 ** 0.5)


def _mesh():
    return plsc.VectorSubcoreMesh(
        core_axis_name="c", subcore_axis_name="s", num_cores=NC, num_subcores=NS)


def _wid():
    return lax.axis_index("s") * NC + lax.axis_index("c")


def _zero_nd(zf_hbm, acc_sp, sid, rps, rextra):
    pltpu.sync_copy(zf_hbm.at[pl.ds(sid * rps, rps)],
                    acc_sp.at[pl.ds(sid * rps, rps)])
    if rextra:
        @pl.when(sid == NS - 1)
        def _():
            pltpu.sync_copy(zf_hbm.at[pl.ds(NS * rps, rextra)],
                            acc_sp.at[pl.ds(NS * rps, rextra)])


def _dump_nd(acc_sp, out, cid, sid, rps, rextra):
    pltpu.sync_copy(acc_sp.at[pl.ds(sid * rps, rps)],
                    out.at[cid, pl.ds(sid * rps, rps)])
    if rextra:
        @pl.when(sid == NS - 1)
        def _():
            pltpu.sync_copy(acc_sp.at[pl.ds(NS * rps, rextra)],
                            out.at[cid, pl.ds(NS * rps, rextra)])


def _sc_attn_w(q, k, src, dst, zeros_n, n, e):
    ep = e // NW
    nch = ep // CHUNK
    assert ep % CHUNK == 0 and CHUNK % LANES == 0

    def body(q_hbm, k_hbm, src_hbm, dst_hbm, zn_hbm,
             w_out, den_out, cnt_out, *scr):
        srcv = list(scr[0:4])
        dstv = list(scr[4:8])
        qb = list(scr[8:10])
        kb = list(scr[10:12])
        wb = list(scr[12:14])
        onesbuf = scr[14]
        den_sp, cnt_sp = scr[15], scr[16]
        semi = list(scr[17:21])
        semg = list(scr[21:23])

        cid = lax.axis_index("c")
        sid = lax.axis_index("s")
        wid = _wid()
        base0 = wid * ep

        nblk = n // CHUNK
        tpb = (nblk + NS - 1) // NS
        pltpu.sync_copy(zn_hbm.at[pl.ds(0, CHUNK)], wb[0])

        def zblk(t, c):
            idx = sid + NS * t
            @pl.when(idx < nblk)
            def _():
                pltpu.sync_copy(wb[0], den_sp.at[pl.ds(idx * CHUNK, CHUNK)])
                pltpu.sync_copy(wb[0], cnt_sp.at[pl.ds(idx * CHUNK, CHUNK)])
            return c
        lax.fori_loop(0, tpb, zblk, 0)
        for g in range(CHUNK // LANES):
            onesbuf[pl.ds(g * LANES, LANES)] = jnp.ones((LANES,), jnp.float32)
        plsc.subcore_barrier()

        pltpu.async_copy(src_hbm.at[pl.ds(base0, CHUNK)], srcv[0], semi[0])
        pltpu.async_copy(dst_hbm.at[pl.ds(base0, CHUNK)], dstv[0], semi[0])
        pltpu.async_copy(src_hbm.at[pl.ds(base0 + CHUNK, CHUNK)], srcv[1],
                         semi[1])
        pltpu.async_copy(dst_hbm.at[pl.ds(base0 + CHUNK, CHUNK)], dstv[1],
                         semi[1])
        pltpu.make_async_copy(src_hbm.at[pl.ds(base0, CHUNK)], srcv[0],
                              semi[0]).wait()
        pltpu.make_async_copy(dst_hbm.at[pl.ds(base0, CHUNK)], dstv[0],
                              semi[0]).wait()
        pltpu.async_copy(q_hbm.at[dstv[0]], qb[0], semg[0])
        pltpu.async_copy(k_hbm.at[srcv[0]], kb[0], semg[0])

        lanes = lax.iota(jnp.int32, LANES)

        def slot(ci, u):
            b = u % 2
            u1, b1 = (u + 1) % 4, (u + 1) % 2
            u2 = (u + 2) % 4
            live = ci < nch
            base = base0 + ci * CHUNK

            @pl.when(live)
            def _():
                pltpu.make_async_copy(q_hbm.at[dstv[u]], qb[b],
                                      semg[b]).wait()
                pltpu.make_async_copy(k_hbm.at[srcv[u]], kb[b],
                                      semg[b]).wait()

            @pl.when(ci + 1 < nch)
            def _():
                base1 = base0 + (ci + 1) * CHUNK
                pltpu.make_async_copy(src_hbm.at[pl.ds(base1, CHUNK)],
                                      srcv[u1], semi[u1]).wait()
                pltpu.make_async_copy(dst_hbm.at[pl.ds(base1, CHUNK)],
                                      dstv[u1], semi[u1]).wait()
                pltpu.async_copy(q_hbm.at[dstv[u1]], qb[b1], semg[b1])
                pltpu.async_copy(k_hbm.at[srcv[u1]], kb[b1], semg[b1])

            @pl.when(live)
            def _():
                for g in range(CHUNK // LANES):
                    rows = g * LANES + lanes

                    def dot_body(dd, acc):
                        for uu in range(4):
                            col = (lanes + (dd * 4 + uu)) & 127
                            qv = plsc.load_gather(qb[b], [rows, col])
                            kv = plsc.load_gather(kb[b], [rows, col])
                            acc = acc + qv * kv
                        return acc
                    acc = lax.fori_loop(0, 32, dot_body,
                                        jnp.zeros((LANES,), jnp.float32))
                    wb[b][pl.ds(g * LANES, LANES)] = (
                        jnp.exp(acc * _INV_SCALE))
                pltpu.sync_copy(wb[b], w_out.at[pl.ds(base, CHUNK)])
                pltpu.sync_copy(wb[b], den_sp.at[dstv[u]], add=True)
                pltpu.sync_copy(onesbuf, cnt_sp.at[dstv[u]], add=True)

            @pl.when(ci + 2 < nch)
            def _():
                base2 = base0 + (ci + 2) * CHUNK
                pltpu.async_copy(src_hbm.at[pl.ds(base2, CHUNK)], srcv[u2],
                                 semi[u2])
                pltpu.async_copy(dst_hbm.at[pl.ds(base2, CHUNK)], dstv[u2],
                                 semi[u2])

        def group(gi, c):
            for u in range(4):
                slot(gi * 4 + u, u)
            return c
        lax.fori_loop(0, (nch + 3) // 4, group, 0)

        plsc.subcore_barrier()

        def dblk(t, c):
            idx = sid + NS * t
            @pl.when(idx < nblk)
            def _():
                pltpu.sync_copy(den_sp.at[pl.ds(idx * CHUNK, CHUNK)], wb[0])
                pltpu.sync_copy(
                    wb[0], den_out.at[pl.ds(cid * n + idx * CHUNK, CHUNK)])
                pltpu.sync_copy(cnt_sp.at[pl.ds(idx * CHUNK, CHUNK)], wb[1])
                pltpu.sync_copy(
                    wb[1], cnt_out.at[pl.ds(cid * n + idx * CHUNK, CHUNK)])
            return c
        lax.fori_loop(0, tpb, dblk, 0)

    fn = pl.kernel(
        body,
        out_type=(jax.ShapeDtypeStruct((e,), jnp.float32),
                  jax.ShapeDtypeStruct((NC * n,), jnp.float32),
                  jax.ShapeDtypeStruct((NC * n,), jnp.float32)),
        mesh=_mesh(),
        compiler_params=pltpu.CompilerParams(needs_layout_passes=False),
        scratch_types=(
            [pltpu.VMEM((CHUNK,), jnp.int32) for _ in range(8)]
            + [pltpu.VMEM((CHUNK, 128), jnp.float32) for _ in range(4)]
            + [pltpu.VMEM((CHUNK,), jnp.float32) for _ in range(3)]
            + [pltpu.VMEM_SHARED((n,), jnp.float32) for _ in range(2)]
            + [pltpu.SemaphoreType.DMA for _ in range(6)]
        ),
    )
    return fn(q, k, src, dst, zeros_n)


def _sc_scatter(h, src, dst, zeros_nd, n, e, w=None):
    ep = e // NW
    nch = ep // CHUNK
    rps = (n // NS) & ~7
    rextra = n - NS * rps
    scaled = w is not None

    def body(*args):
        if scaled:
            (h_hbm, src_hbm, dst_hbm, zf_hbm, w_hbm, s_out, *scr) = args
        else:
            (h_hbm, src_hbm, dst_hbm, zf_hbm, s_out, *scr) = args
        srcv = list(scr[0:4])
        dstv = list(scr[4:8])
        buf = list(scr[8:10])
        s_sp = scr[10]
        semi = list(scr[11:15])
        semg = list(scr[15:17])
        wv = list(scr[17:21]) if scaled else None
        buf2 = list(scr[21:23]) if scaled else buf

        cid = lax.axis_index("c")
        sid = lax.axis_index("s")
        wid = _wid()
        base0 = wid * ep

        _zero_nd(zf_hbm, s_sp, sid, rps, rextra)
        plsc.subcore_barrier()

        def fetch_idx(ci, u):
            base = base0 + ci * CHUNK
            pltpu.async_copy(src_hbm.at[pl.ds(base, CHUNK)], srcv[u], semi[u])
            pltpu.async_copy(dst_hbm.at[pl.ds(base, CHUNK)], dstv[u], semi[u])
            if scaled:
                pltpu.async_copy(w_hbm.at[pl.ds(base, CHUNK)], wv[u], semi[u])

        def wait_idx(ci, u):
            base = base0 + ci * CHUNK
            pltpu.make_async_copy(src_hbm.at[pl.ds(base, CHUNK)], srcv[u],
                                  semi[u]).wait()
            pltpu.make_async_copy(dst_hbm.at[pl.ds(base, CHUNK)], dstv[u],
                                  semi[u]).wait()
            if scaled:
                pltpu.make_async_copy(w_hbm.at[pl.ds(base, CHUNK)], wv[u],
                                      semi[u]).wait()

        fetch_idx(0, 0)
        fetch_idx(1, 1)
        wait_idx(0, 0)
        pltpu.async_copy(h_hbm.at[srcv[0]], buf[0], semg[0])

        lanes = lax.iota(jnp.int32, LANES)

        def slot(ci, u):
            b = u % 2
            u1, b1 = (u + 1) % 4, (u + 1) % 2
            u2 = (u + 2) % 4
            live = ci < nch

            @pl.when(live)
            def _():
                pltpu.make_async_copy(h_hbm.at[srcv[u]], buf[b],
                                      semg[b]).wait()

            @pl.when(ci + 1 < nch)
            def _():
                wait_idx(ci + 1, u1)
                pltpu.async_copy(h_hbm.at[srcv[u1]], buf[b1], semg[b1])

            @pl.when(live)
            def _():
                if scaled:
                    for g in range(CHUNK // LANES):
                        rows = g * LANES + lanes
                        w16 = wv[u][pl.ds(g * LANES, LANES)]

                        def scale_body(dd, c):
                            for uu in range(4):
                                col = (lanes + (dd * 4 + uu)) & 127
                                vvv = plsc.load_gather(buf[b], [rows, col])
                                plsc.store_scatter(buf2[b], [rows, col],
                                                   vvv * w16)
                            return c
                        lax.fori_loop(0, 32, scale_body, 0)
                pltpu.sync_copy(buf2[b], s_sp.at[dstv[u]], add=True)

            @pl.when(ci + 2 < nch)
            def _():
                fetch_idx(ci + 2, u2)

        def group(gi, c):
            for u in range(4):
                slot(gi * 4 + u, u)
            return c
        lax.fori_loop(0, (nch + 3) // 4, group, 0)

        plsc.subcore_barrier()
        _dump_nd(s_sp, s_out, cid, sid, rps, rextra)

    fn = pl.kernel(
        body,
        out_type=jax.ShapeDtypeStruct((NC, n, 128), jnp.float32),
        mesh=_mesh(),
        compiler_params=pltpu.CompilerParams(needs_layout_passes=False),
        scratch_types=(
            [pltpu.VMEM((CHUNK,), jnp.int32) for _ in range(8)]
            + [pltpu.VMEM((CHUNK, 128), jnp.float32) for _ in range(2)]
            + [pltpu.VMEM_SHARED((n, 128), jnp.float32)]
            + [pltpu.SemaphoreType.DMA for _ in range(6)]
            + ([pltpu.VMEM((CHUNK,), jnp.float32) for _ in range(4)]
               + [pltpu.VMEM((CHUNK, 128), jnp.float32) for _ in range(2)]
               if scaled else [])
        ),
    )
    if scaled:
        return fn(h, src, dst, zeros_nd, w)
    return fn(h, src, dst, zeros_nd)


_ROWS = 2000


def _tc_qkvs(x, wc, bc, n):
    grid = n // _ROWS

    def body(x_ref, w_ref, b_ref, o_ref):
        o_ref[...] = jnp.dot(x_ref[...], w_ref[...],
                             preferred_element_type=jnp.float32) + b_ref[...]

    return pl.pallas_call(
        body,
        grid=(grid,),
        in_specs=[
            pl.BlockSpec((_ROWS, 128), lambda i: (i, 0)),
            pl.BlockSpec((128, 512), lambda i: (0, 0)),
            pl.BlockSpec((1, 512), lambda i: (0, 0)),
        ],
        out_specs=pl.BlockSpec((_ROWS, 512), lambda i: (i, 0)),
        out_shape=jax.ShapeDtypeStruct((n, 512), jnp.float32),
    )(x, wc, bc)


def _tc_h(zp, denp, s, n):
    grid = n // _ROWS

    def body(z_ref, d_ref, s_ref, o_ref):
        z = z_ref[0] + z_ref[1]
        dsl = d_ref[0]
        den = dsl[0] + dsl[1] + 1e-16
        o_ref[...] = jnp.maximum(z / den[:, None] + s_ref[...], 0.0)

    return pl.pallas_call(
        body,
        grid=(grid,),
        in_specs=[
            pl.BlockSpec((NC, _ROWS, 128), lambda i: (0, i, 0)),
            pl.BlockSpec((1, NC, _ROWS), lambda i: (i, 0, 0)),
            pl.BlockSpec((_ROWS, 128), lambda i: (i, 0)),
        ],
        out_specs=pl.BlockSpec((_ROWS, 128), lambda i: (i, 0)),
        out_shape=jax.ShapeDtypeStruct((n, 128), jnp.float32),
    )(zp, denp, s)


def _tc_sage(sp, cntp, h, wl, wr, bf, n):
    grid = n // _ROWS

    def body(s_ref, c_ref, h_ref, wl_ref, wr_ref, b_ref, o_ref):
        csl = c_ref[0]
        cnt = jnp.maximum(csl[0] + csl[1], 1.0)
        mean = (s_ref[0] + s_ref[1]) / cnt[:, None]
        out = (jnp.dot(mean, wl_ref[...], preferred_element_type=jnp.float32)
               + jnp.dot(h_ref[...], wr_ref[...],
                         preferred_element_type=jnp.float32)
               + b_ref[...])
        o_ref[...] = jnp.maximum(out, 0.0)

    return pl.pallas_call(
        body,
        grid=(grid,),
        in_specs=[
            pl.BlockSpec((NC, _ROWS, 128), lambda i: (0, i, 0)),
            pl.BlockSpec((1, NC, _ROWS), lambda i: (i, 0, 0)),
            pl.BlockSpec((_ROWS, 128), lambda i: (i, 0)),
            pl.BlockSpec((128, 128), lambda i: (0, 0)),
            pl.BlockSpec((128, 128), lambda i: (0, 0)),
            pl.BlockSpec((1, 128), lambda i: (0, 0)),
        ],
        out_specs=pl.BlockSpec((_ROWS, 128), lambda i: (i, 0)),
        out_shape=jax.ShapeDtypeStruct((n, 128), jnp.float32),
    )(sp, cntp, h, wl, wr, bf)


def kernel(x, edge_index, params):
    n, d = x.shape
    e = edge_index.shape[1]
    assert d == 128

    src = edge_index[0].astype(jnp.int32)
    dst = edge_index[1].astype(jnp.int32)

    wc = jnp.concatenate([params["Wq"].T, params["Wk"].T,
                          params["Wv"].T, params["Wskip"].T], axis=1)
    bc = jnp.concatenate([params["bq"], params["bk"],
                          params["bv"], params["bskip"]])[None, :]
    a = jax.nn.sigmoid(params["res_alpha"])
    eye = jnp.eye(128, dtype=jnp.float32)
    wls, wrs, bfs = [], [], []
    for i in range(3):
        sbn = params["bn_g%d" % i] * lax.rsqrt(params["bn_v%d" % i] + 1e-5)
        wls.append(a * (params["W_l%d" % i].T * sbn[None, :]))
        wrs.append(a * (params["W_r%d" % i].T * sbn[None, :]) + (1.0 - a) * eye)
        bfs.append((a * ((params["b_l%d" % i] - params["bn_m%d" % i]) * sbn
                         + params["bn_b%d" % i]))[None, :])

    zeros_nd = jnp.zeros((n, 128), jnp.float32)
    zeros_n = jnp.zeros((n,), jnp.float32)

    qkvs = _tc_qkvs(x, wc, bc, n)
    q = qkvs[:, 0:128]
    k = qkvs[:, 128:256]
    v = qkvs[:, 256:384]
    s = qkvs[:, 384:512]

    we, denp, cntp = _sc_attn_w(q, k, src, dst, zeros_n, n, e)
    zp = _sc_scatter(v, src, dst, zeros_nd, n, e, w=we)

    grid = n // _ROWS
    denp = denp.reshape(NC, grid, _ROWS).transpose(1, 0, 2)
    cntp = cntp.reshape(NC, grid, _ROWS).transpose(1, 0, 2)

    h = _tc_h(zp, denp, s, n)
    for i in range(3):
        sp = _sc_scatter(h, src, dst, zeros_nd, n, e)
        h = _tc_sage(sp, cntp, h, wls[i], wrs[i], bfs[i], n)
    return h


# speedup vs baseline: 9.1979x; 1.0404x over previous
import jax
import jax.numpy as jnp
from jax import lax
from jax.experimental import pallas as pl
from jax.experimental.pallas import tpu as pltpu
from jax.experimental.pallas import tpu_sc as plsc

NC = 2
NS = 16
NW = NC * NS
LANES = 16
CHUNK = 80

_INV_SCALE = 1.0 / (128.0 ** 0.5)


def _mesh():
    return plsc.VectorSubcoreMesh(
        core_axis_name="c", subcore_axis_name="s", num_cores=NC, num_subcores=NS)


def _wid():
    return lax.axis_index("s") * NC + lax.axis_index("c")


def _zero_nd(zf_hbm, acc_sp, sid, rps, rextra):
    pltpu.sync_copy(zf_hbm.at[pl.ds(sid * rps, rps)],
                    acc_sp.at[pl.ds(sid * rps, rps)])
    if rextra:
        @pl.when(sid == NS - 1)
        def _():
            pltpu.sync_copy(zf_hbm.at[pl.ds(NS * rps, rextra)],
                            acc_sp.at[pl.ds(NS * rps, rextra)])


def _dump_nd(acc_sp, out, cid, sid, rps, rextra):
    pltpu.sync_copy(acc_sp.at[pl.ds(sid * rps, rps)],
                    out.at[cid, pl.ds(sid * rps, rps)])
    if rextra:
        @pl.when(sid == NS - 1)
        def _():
            pltpu.sync_copy(acc_sp.at[pl.ds(NS * rps, rextra)],
                            out.at[cid, pl.ds(NS * rps, rextra)])


def _sc_attn_w(q, k, src, dst, n, e):
    ep = e // NW
    nch = ep // CHUNK
    assert ep % CHUNK == 0 and CHUNK % LANES == 0

    def body(q_hbm, k_hbm, src_hbm, dst_hbm,
             w_out, den_out, cnt_out, *scr):
        srcv = list(scr[0:4])
        dstv = list(scr[4:8])
        qb = list(scr[8:10])
        kb = list(scr[10:12])
        wb = list(scr[12:14])
        den_l, cnt_l = scr[14], scr[15]
        semi = list(scr[16:20])
        semg = list(scr[20:22])
        semw = list(scr[22:24])

        wid = _wid()
        base0 = wid * ep

        def zloc(i, c):
            zv = jnp.zeros((LANES,), jnp.float32)
            den_l[pl.ds(i * LANES, LANES)] = zv
            cnt_l[pl.ds(i * LANES, LANES)] = zv
            return c
        lax.fori_loop(0, n // LANES, zloc, 0)

        pltpu.async_copy(src_hbm.at[pl.ds(base0, CHUNK)], srcv[0], semi[0])
        pltpu.async_copy(dst_hbm.at[pl.ds(base0, CHUNK)], dstv[0], semi[0])
        pltpu.async_copy(src_hbm.at[pl.ds(base0 + CHUNK, CHUNK)], srcv[1],
                         semi[1])
        pltpu.async_copy(dst_hbm.at[pl.ds(base0 + CHUNK, CHUNK)], dstv[1],
                         semi[1])
        pltpu.make_async_copy(src_hbm.at[pl.ds(base0, CHUNK)], srcv[0],
                              semi[0]).wait()
        pltpu.make_async_copy(dst_hbm.at[pl.ds(base0, CHUNK)], dstv[0],
                              semi[0]).wait()
        pltpu.async_copy(q_hbm.at[dstv[0]], qb[0], semg[0])
        pltpu.async_copy(k_hbm.at[srcv[0]], kb[0], semg[0])

        lanes = lax.iota(jnp.int32, LANES)

        def slot(ci, u):
            b = u % 2
            u1, b1 = (u + 1) % 4, (u + 1) % 2
            u2 = (u + 2) % 4
            live = ci < nch
            base = base0 + ci * CHUNK

            @pl.when(live)
            def _():
                pltpu.make_async_copy(q_hbm.at[dstv[u]], qb[b],
                                      semg[b]).wait()
                pltpu.make_async_copy(k_hbm.at[srcv[u]], kb[b],
                                      semg[b]).wait()

            @pl.when(jnp.logical_and(live, ci >= 2))
            def _():
                basep = base0 + (ci - 2) * CHUNK
                pltpu.make_async_copy(wb[b], w_out.at[pl.ds(basep, CHUNK)],
                                      semw[b]).wait()

            @pl.when(ci + 1 < nch)
            def _():
                base1 = base0 + (ci + 1) * CHUNK
                pltpu.make_async_copy(src_hbm.at[pl.ds(base1, CHUNK)],
                                      srcv[u1], semi[u1]).wait()
                pltpu.make_async_copy(dst_hbm.at[pl.ds(base1, CHUNK)],
                                      dstv[u1], semi[u1]).wait()
                pltpu.async_copy(q_hbm.at[dstv[u1]], qb[b1], semg[b1])
                pltpu.async_copy(k_hbm.at[srcv[u1]], kb[b1], semg[b1])

            @pl.when(live)
            def _():
                ones16 = jnp.ones((LANES,), jnp.float32)
                for g in range(CHUNK // LANES):
                    rows = g * LANES + lanes
                    dstg = dstv[u][pl.ds(g * LANES, LANES)]

                    def dot_body(dd, acc):
                        for uu in range(4):
                            col = (lanes + (dd * 4 + uu)) & 127
                            qv = plsc.load_gather(qb[b], [rows, col])
                            kv = plsc.load_gather(kb[b], [rows, col])
                            acc = acc + qv * kv
                        return acc
                    acc = lax.fori_loop(0, 32, dot_body,
                                        jnp.zeros((LANES,), jnp.float32))
                    w16 = jnp.exp(acc * _INV_SCALE)
                    wb[b][pl.ds(g * LANES, LANES)] = w16
                    plsc.addupdate_scatter(den_l, [dstg], w16)
                    plsc.addupdate_scatter(cnt_l, [dstg], ones16)
                pltpu.async_copy(wb[b], w_out.at[pl.ds(base, CHUNK)], semw[b])

            @pl.when(ci + 2 < nch)
            def _():
                base2 = base0 + (ci + 2) * CHUNK
                pltpu.async_copy(src_hbm.at[pl.ds(base2, CHUNK)], srcv[u2],
                                 semi[u2])
                pltpu.async_copy(dst_hbm.at[pl.ds(base2, CHUNK)], dstv[u2],
                                 semi[u2])

        def group(gi, c):
            for u in range(4):
                slot(gi * 4 + u, u)
            return c
        lax.fori_loop(0, (nch + 3) // 4, group, 0)

        for cl in (nch - 2, nch - 1):
            basel = base0 + cl * CHUNK
            pltpu.make_async_copy(wb[cl % 2], w_out.at[pl.ds(basel, CHUNK)],
                                  semw[cl % 2]).wait()
        pltpu.sync_copy(den_l, den_out.at[pl.ds(wid * n, n)])
        pltpu.sync_copy(cnt_l, cnt_out.at[pl.ds(wid * n, n)])

    fn = pl.kernel(
        body,
        out_type=(jax.ShapeDtypeStruct((e,), jnp.float32),
                  jax.ShapeDtypeStruct((NW * n,), jnp.float32),
                  jax.ShapeDtypeStruct((NW * n,), jnp.float32)),
        mesh=_mesh(),
        compiler_params=pltpu.CompilerParams(needs_layout_passes=False),
        scratch_types=(
            [pltpu.VMEM((CHUNK,), jnp.int32) for _ in range(8)]
            + [pltpu.VMEM((CHUNK, 128), jnp.float32) for _ in range(4)]
            + [pltpu.VMEM((CHUNK,), jnp.float32) for _ in range(2)]
            + [pltpu.VMEM((n,), jnp.float32) for _ in range(2)]
            + [pltpu.SemaphoreType.DMA for _ in range(8)]
        ),
    )
    return fn(q, k, src, dst)


def _sc_scatter(h, src, dst, zeros_nd, n, e, w=None):
    ep = e // NW
    nch = ep // CHUNK
    rps = (n // NS) & ~7
    rextra = n - NS * rps
    scaled = w is not None

    def body(*args):
        if scaled:
            (h_hbm, src_hbm, dst_hbm, zf_hbm, w_hbm, s_out, *scr) = args
        else:
            (h_hbm, src_hbm, dst_hbm, zf_hbm, s_out, *scr) = args
        srcv = list(scr[0:4])
        dstv = list(scr[4:8])
        buf = list(scr[8:10])
        s_sp = scr[10]
        semi = list(scr[11:15])
        semg = list(scr[15:17])
        wv = list(scr[17:21]) if scaled else None
        buf2 = list(scr[21:23]) if scaled else buf

        cid = lax.axis_index("c")
        sid = lax.axis_index("s")
        wid = _wid()
        base0 = wid * ep

        _zero_nd(zf_hbm, s_sp, sid, rps, rextra)
        plsc.subcore_barrier()

        def fetch_idx(ci, u):
            base = base0 + ci * CHUNK
            pltpu.async_copy(src_hbm.at[pl.ds(base, CHUNK)], srcv[u], semi[u])
            pltpu.async_copy(dst_hbm.at[pl.ds(base, CHUNK)], dstv[u], semi[u])
            if scaled:
                pltpu.async_copy(w_hbm.at[pl.ds(base, CHUNK)], wv[u], semi[u])

        def wait_idx(ci, u):
            base = base0 + ci * CHUNK
            pltpu.make_async_copy(src_hbm.at[pl.ds(base, CHUNK)], srcv[u],
                                  semi[u]).wait()
            pltpu.make_async_copy(dst_hbm.at[pl.ds(base, CHUNK)], dstv[u],
                                  semi[u]).wait()
            if scaled:
                pltpu.make_async_copy(w_hbm.at[pl.ds(base, CHUNK)], wv[u],
                                      semi[u]).wait()

        fetch_idx(0, 0)
        fetch_idx(1, 1)
        wait_idx(0, 0)
        pltpu.async_copy(h_hbm.at[srcv[0]], buf[0], semg[0])

        lanes = lax.iota(jnp.int32, LANES)

        def slot(ci, u):
            b = u % 2
            u1, b1 = (u + 1) % 4, (u + 1) % 2
            u2 = (u + 2) % 4
            live = ci < nch

            @pl.when(live)
            def _():
                pltpu.make_async_copy(h_hbm.at[srcv[u]], buf[b],
                                      semg[b]).wait()

            @pl.when(ci + 1 < nch)
            def _():
                wait_idx(ci + 1, u1)
                pltpu.async_copy(h_hbm.at[srcv[u1]], buf[b1], semg[b1])

            @pl.when(live)
            def _():
                if scaled:
                    def scale_edge(ei, c):
                        wsp = plsc.load_gather(
                            wv[u], [jnp.full((LANES,), ei, jnp.int32)])
                        for j in range(8):
                            seg = buf[b][ei, pl.ds(j * LANES, LANES)]
                            buf2[b][ei, pl.ds(j * LANES, LANES)] = seg * wsp
                        return c
                    lax.fori_loop(0, CHUNK, scale_edge, 0)
                pltpu.sync_copy(buf2[b], s_sp.at[dstv[u]], add=True)

            @pl.when(ci + 2 < nch)
            def _():
                fetch_idx(ci + 2, u2)

        def group(gi, c):
            for u in range(4):
                slot(gi * 4 + u, u)
            return c
        lax.fori_loop(0, (nch + 3) // 4, group, 0)

        plsc.subcore_barrier()
        _dump_nd(s_sp, s_out, cid, sid, rps, rextra)

    fn = pl.kernel(
        body,
        out_type=jax.ShapeDtypeStruct((NC, n, 128), jnp.float32),
        mesh=_mesh(),
        compiler_params=pltpu.CompilerParams(needs_layout_passes=False),
        scratch_types=(
            [pltpu.VMEM((CHUNK,), jnp.int32) for _ in range(8)]
            + [pltpu.VMEM((CHUNK, 128), jnp.float32) for _ in range(2)]
            + [pltpu.VMEM_SHARED((n, 128), jnp.float32)]
            + [pltpu.SemaphoreType.DMA for _ in range(6)]
            + ([pltpu.VMEM((CHUNK,), jnp.float32) for _ in range(4)]
               + [pltpu.VMEM((CHUNK, 128), jnp.float32) for _ in range(2)]
               if scaled else [])
        ),
    )
    if scaled:
        return fn(h, src, dst, zeros_nd, w)
    return fn(h, src, dst, zeros_nd)


_ROWS = 2000


def _tc_qkvs(x, wc, bc, n):
    grid = n // _ROWS

    def body(x_ref, w_ref, b_ref, o_ref):
        o_ref[...] = jnp.dot(x_ref[...], w_ref[...],
                             preferred_element_type=jnp.float32) + b_ref[...]

    return pl.pallas_call(
        body,
        grid=(grid,),
        in_specs=[
            pl.BlockSpec((_ROWS, 128), lambda i: (i, 0)),
            pl.BlockSpec((128, 512), lambda i: (0, 0)),
            pl.BlockSpec((1, 512), lambda i: (0, 0)),
        ],
        out_specs=pl.BlockSpec((_ROWS, 512), lambda i: (i, 0)),
        out_shape=jax.ShapeDtypeStruct((n, 512), jnp.float32),
    )(x, wc, bc)


def _tc_h(zp, denp, s, n):
    grid = n // _ROWS

    def body(z_ref, d_ref, s_ref, o_ref):
        z = z_ref[0] + z_ref[1]
        den = jnp.sum(d_ref[0], axis=0) + 1e-16
        o_ref[...] = jnp.maximum(z / den[:, None] + s_ref[...], 0.0)

    return pl.pallas_call(
        body,
        grid=(grid,),
        in_specs=[
            pl.BlockSpec((NC, _ROWS, 128), lambda i: (0, i, 0)),
            pl.BlockSpec((1, NW, _ROWS), lambda i: (i, 0, 0)),
            pl.BlockSpec((_ROWS, 128), lambda i: (i, 0)),
        ],
        out_specs=pl.BlockSpec((_ROWS, 128), lambda i: (i, 0)),
        out_shape=jax.ShapeDtypeStruct((n, 128), jnp.float32),
    )(zp, denp, s)


def _tc_sage(sp, cntp, h, wl, wr, bf, n):
    grid = n // _ROWS

    def body(s_ref, c_ref, h_ref, wl_ref, wr_ref, b_ref, o_ref):
        cnt = jnp.maximum(jnp.sum(c_ref[0], axis=0), 1.0)
        mean = (s_ref[0] + s_ref[1]) / cnt[:, None]
        out = (jnp.dot(mean, wl_ref[...], preferred_element_type=jnp.float32)
               + jnp.dot(h_ref[...], wr_ref[...],
                         preferred_element_type=jnp.float32)
               + b_ref[...])
        o_ref[...] = jnp.maximum(out, 0.0)

    return pl.pallas_call(
        body,
        grid=(grid,),
        in_specs=[
            pl.BlockSpec((NC, _ROWS, 128), lambda i: (0, i, 0)),
            pl.BlockSpec((1, NW, _ROWS), lambda i: (i, 0, 0)),
            pl.BlockSpec((_ROWS, 128), lambda i: (i, 0)),
            pl.BlockSpec((128, 128), lambda i: (0, 0)),
            pl.BlockSpec((128, 128), lambda i: (0, 0)),
            pl.BlockSpec((1, 128), lambda i: (0, 0)),
        ],
        out_specs=pl.BlockSpec((_ROWS, 128), lambda i: (i, 0)),
        out_shape=jax.ShapeDtypeStruct((n, 128), jnp.float32),
    )(sp, cntp, h, wl, wr, bf)


def kernel(x, edge_index, params):
    n, d = x.shape
    e = edge_index.shape[1]
    assert d == 128

    src = edge_index[0].astype(jnp.int32)
    dst = edge_index[1].astype(jnp.int32)

    wc = jnp.concatenate([params["Wq"].T, params["Wk"].T,
                          params["Wv"].T, params["Wskip"].T], axis=1)
    bc = jnp.concatenate([params["bq"], params["bk"],
                          params["bv"], params["bskip"]])[None, :]
    a = jax.nn.sigmoid(params["res_alpha"])
    eye = jnp.eye(128, dtype=jnp.float32)
    wls, wrs, bfs = [], [], []
    for i in range(3):
        sbn = params["bn_g%d" % i] * lax.rsqrt(params["bn_v%d" % i] + 1e-5)
        wls.append(a * (params["W_l%d" % i].T * sbn[None, :]))
        wrs.append(a * (params["W_r%d" % i].T * sbn[None, :]) + (1.0 - a) * eye)
        bfs.append((a * ((params["b_l%d" % i] - params["bn_m%d" % i]) * sbn
                         + params["bn_b%d" % i]))[None, :])

    zeros_nd = jnp.zeros((n, 128), jnp.float32)

    qkvs = _tc_qkvs(x, wc, bc, n)
    q = qkvs[:, 0:128]
    k = qkvs[:, 128:256]
    v = qkvs[:, 256:384]
    s = qkvs[:, 384:512]

    we, denp, cntp = _sc_attn_w(q, k, src, dst, n, e)
    zp = _sc_scatter(v, src, dst, zeros_nd, n, e, w=we)

    grid = n // _ROWS
    denp = denp.reshape(NW, grid, _ROWS).transpose(1, 0, 2)
    cntp = cntp.reshape(NW, grid, _ROWS).transpose(1, 0, 2)

    h = _tc_h(zp, denp, s, n)
    for i in range(3):
        sp = _sc_scatter(h, src, dst, zeros_nd, n, e)
        h = _tc_sage(sp, cntp, h, wls[i], wrs[i], bfs[i], n)
    return h


# speedup vs baseline: 9.2280x; 1.0033x over previous
import jax
import jax.numpy as jnp
from jax import lax
from jax.experimental import pallas as pl
from jax.experimental.pallas import tpu as pltpu
from jax.experimental.pallas import tpu_sc as plsc

NC = 2
NS = 16
NW = NC * NS
LANES = 16
CHUNK = 80

_INV_SCALE = 1.0 / (128.0 ** 0.5)


def _mesh():
    return plsc.VectorSubcoreMesh(
        core_axis_name="c", subcore_axis_name="s", num_cores=NC, num_subcores=NS)


def _wid():
    return lax.axis_index("s") * NC + lax.axis_index("c")


def _zero_nd(zf_hbm, acc_sp, sid, rps, rextra):
    pltpu.sync_copy(zf_hbm.at[pl.ds(sid * rps, rps)],
                    acc_sp.at[pl.ds(sid * rps, rps)])
    if rextra:
        @pl.when(sid == NS - 1)
        def _():
            pltpu.sync_copy(zf_hbm.at[pl.ds(NS * rps, rextra)],
                            acc_sp.at[pl.ds(NS * rps, rextra)])


def _dump_nd(acc_sp, out, cid, sid, rps, rextra):
    pltpu.sync_copy(acc_sp.at[pl.ds(sid * rps, rps)],
                    out.at[cid, pl.ds(sid * rps, rps)])
    if rextra:
        @pl.when(sid == NS - 1)
        def _():
            pltpu.sync_copy(acc_sp.at[pl.ds(NS * rps, rextra)],
                            out.at[cid, pl.ds(NS * rps, rextra)])


def _sc_attn_w(q, k, src, dst, n, e):
    ep = e // NW
    nch = ep // CHUNK
    assert ep % CHUNK == 0 and CHUNK % LANES == 0

    def body(q_hbm, k_hbm, src_hbm, dst_hbm,
             w_out, den_out, cnt_out, *scr):
        srcv = list(scr[0:4])
        dstv = list(scr[4:8])
        qb = list(scr[8:10])
        kb = list(scr[10:12])
        wb = list(scr[12:14])
        den_l, cnt_l = scr[14], scr[15]
        semi = list(scr[16:20])
        semg = list(scr[20:22])
        semw = list(scr[22:24])

        wid = _wid()
        base0 = wid * ep

        def zloc(i, c):
            zv = jnp.zeros((LANES,), jnp.float32)
            den_l[pl.ds(i * LANES, LANES)] = zv
            cnt_l[pl.ds(i * LANES, LANES)] = zv
            return c
        lax.fori_loop(0, n // LANES, zloc, 0)

        pltpu.async_copy(src_hbm.at[pl.ds(base0, CHUNK)], srcv[0], semi[0])
        pltpu.async_copy(dst_hbm.at[pl.ds(base0, CHUNK)], dstv[0], semi[0])
        pltpu.async_copy(src_hbm.at[pl.ds(base0 + CHUNK, CHUNK)], srcv[1],
                         semi[1])
        pltpu.async_copy(dst_hbm.at[pl.ds(base0 + CHUNK, CHUNK)], dstv[1],
                         semi[1])
        pltpu.make_async_copy(src_hbm.at[pl.ds(base0, CHUNK)], srcv[0],
                              semi[0]).wait()
        pltpu.make_async_copy(dst_hbm.at[pl.ds(base0, CHUNK)], dstv[0],
                              semi[0]).wait()
        pltpu.async_copy(q_hbm.at[dstv[0]], qb[0], semg[0])
        pltpu.async_copy(k_hbm.at[srcv[0]], kb[0], semg[0])

        lanes = lax.iota(jnp.int32, LANES)

        def slot(ci, u):
            b = u % 2
            u1, b1 = (u + 1) % 4, (u + 1) % 2
            u2 = (u + 2) % 4
            live = ci < nch
            base = base0 + ci * CHUNK

            @pl.when(live)
            def _():
                pltpu.make_async_copy(q_hbm.at[dstv[u]], qb[b],
                                      semg[b]).wait()
                pltpu.make_async_copy(k_hbm.at[srcv[u]], kb[b],
                                      semg[b]).wait()

            @pl.when(jnp.logical_and(live, ci >= 2))
            def _():
                basep = base0 + (ci - 2) * CHUNK
                pltpu.make_async_copy(wb[b], w_out.at[pl.ds(basep, CHUNK)],
                                      semw[b]).wait()

            @pl.when(ci + 1 < nch)
            def _():
                base1 = base0 + (ci + 1) * CHUNK
                pltpu.make_async_copy(src_hbm.at[pl.ds(base1, CHUNK)],
                                      srcv[u1], semi[u1]).wait()
                pltpu.make_async_copy(dst_hbm.at[pl.ds(base1, CHUNK)],
                                      dstv[u1], semi[u1]).wait()
                pltpu.async_copy(q_hbm.at[dstv[u1]], qb[b1], semg[b1])
                pltpu.async_copy(k_hbm.at[srcv[u1]], kb[b1], semg[b1])

            @pl.when(live)
            def _():
                ones16 = jnp.ones((LANES,), jnp.float32)
                for g in range(CHUNK // LANES):
                    rows = g * LANES + lanes
                    dstg = dstv[u][pl.ds(g * LANES, LANES)]

                    def dot_body(dd, acc):
                        for uu in range(4):
                            col = (lanes + (dd * 4 + uu)) & 127
                            qv = plsc.load_gather(qb[b], [rows, col])
                            kv = plsc.load_gather(kb[b], [rows, col])
                            acc = acc + qv * kv
                        return acc
                    acc = lax.fori_loop(0, 32, dot_body,
                                        jnp.zeros((LANES,), jnp.float32))
                    w16 = jnp.exp(acc * _INV_SCALE)
                    wb[b][pl.ds(g * LANES, LANES)] = w16
                    plsc.addupdate_scatter(den_l, [dstg], w16)
                    plsc.addupdate_scatter(cnt_l, [dstg], ones16)
                pltpu.async_copy(wb[b], w_out.at[pl.ds(base, CHUNK)], semw[b])

            @pl.when(ci + 2 < nch)
            def _():
                base2 = base0 + (ci + 2) * CHUNK
                pltpu.async_copy(src_hbm.at[pl.ds(base2, CHUNK)], srcv[u2],
                                 semi[u2])
                pltpu.async_copy(dst_hbm.at[pl.ds(base2, CHUNK)], dstv[u2],
                                 semi[u2])

        def group(gi, c):
            for u in range(4):
                slot(gi * 4 + u, u)
            return c
        lax.fori_loop(0, (nch + 3) // 4, group, 0)

        for cl in (nch - 2, nch - 1):
            basel = base0 + cl * CHUNK
            pltpu.make_async_copy(wb[cl % 2], w_out.at[pl.ds(basel, CHUNK)],
                                  semw[cl % 2]).wait()
        pltpu.sync_copy(den_l, den_out.at[pl.ds(wid * n, n)])
        pltpu.sync_copy(cnt_l, cnt_out.at[pl.ds(wid * n, n)])

    fn = pl.kernel(
        body,
        out_type=(jax.ShapeDtypeStruct((e,), jnp.float32),
                  jax.ShapeDtypeStruct((NW * n,), jnp.float32),
                  jax.ShapeDtypeStruct((NW * n,), jnp.float32)),
        mesh=_mesh(),
        compiler_params=pltpu.CompilerParams(needs_layout_passes=False),
        scratch_types=(
            [pltpu.VMEM((CHUNK,), jnp.int32) for _ in range(8)]
            + [pltpu.VMEM((CHUNK, 128), jnp.float32) for _ in range(4)]
            + [pltpu.VMEM((CHUNK,), jnp.float32) for _ in range(2)]
            + [pltpu.VMEM((n,), jnp.float32) for _ in range(2)]
            + [pltpu.SemaphoreType.DMA for _ in range(8)]
        ),
    )
    return fn(q, k, src, dst)


def _sc_scatter(h, src, dst, zeros_nd, n, e, w=None):
    ep = e // NW
    nch = ep // CHUNK
    rps = (n // NS) & ~7
    rextra = n - NS * rps
    scaled = w is not None

    def body(*args):
        if scaled:
            (h_hbm, src_hbm, dst_hbm, zf_hbm, w_hbm, s_out, *scr) = args
        else:
            (h_hbm, src_hbm, dst_hbm, zf_hbm, s_out, *scr) = args
        srcv = list(scr[0:4])
        dstv = list(scr[4:8])
        buf = list(scr[8:10])
        s_sp = scr[10]
        semi = list(scr[11:15])
        semg = list(scr[15:17])
        wv = list(scr[17:21]) if scaled else None
        buf2 = list(scr[21:23]) if scaled else buf

        cid = lax.axis_index("c")
        sid = lax.axis_index("s")
        wid = _wid()
        base0 = wid * ep

        _zero_nd(zf_hbm, s_sp, sid, rps, rextra)
        plsc.subcore_barrier()

        def fetch_idx(ci, u):
            base = base0 + ci * CHUNK
            pltpu.async_copy(src_hbm.at[pl.ds(base, CHUNK)], srcv[u], semi[u])
            pltpu.async_copy(dst_hbm.at[pl.ds(base, CHUNK)], dstv[u], semi[u])
            if scaled:
                pltpu.async_copy(w_hbm.at[pl.ds(base, CHUNK)], wv[u], semi[u])

        def wait_idx(ci, u):
            base = base0 + ci * CHUNK
            pltpu.make_async_copy(src_hbm.at[pl.ds(base, CHUNK)], srcv[u],
                                  semi[u]).wait()
            pltpu.make_async_copy(dst_hbm.at[pl.ds(base, CHUNK)], dstv[u],
                                  semi[u]).wait()
            if scaled:
                pltpu.make_async_copy(w_hbm.at[pl.ds(base, CHUNK)], wv[u],
                                      semi[u]).wait()

        fetch_idx(0, 0)
        fetch_idx(1, 1)
        wait_idx(0, 0)
        pltpu.async_copy(h_hbm.at[srcv[0]], buf[0], semg[0])

        lanes = lax.iota(jnp.int32, LANES)

        def slot(ci, u):
            b = u % 2
            u1, b1 = (u + 1) % 4, (u + 1) % 2
            u2 = (u + 2) % 4
            live = ci < nch

            @pl.when(live)
            def _():
                pltpu.make_async_copy(h_hbm.at[srcv[u]], buf[b],
                                      semg[b]).wait()

            @pl.when(ci + 1 < nch)
            def _():
                wait_idx(ci + 1, u1)
                pltpu.async_copy(h_hbm.at[srcv[u1]], buf[b1], semg[b1])

            @pl.when(live)
            def _():
                if scaled:
                    def scale_edge(it, c):
                        for jj in range(8):
                            ei = it * 8 + jj
                            wsp = plsc.load_gather(
                                wv[u], [jnp.full((LANES,), ei, jnp.int32)])
                            for j in range(8):
                                seg = buf[b][ei, pl.ds(j * LANES, LANES)]
                                buf2[b][ei, pl.ds(j * LANES, LANES)] = (
                                    seg * wsp)
                        return c
                    lax.fori_loop(0, CHUNK // 8, scale_edge, 0)
                pltpu.sync_copy(buf2[b], s_sp.at[dstv[u]], add=True)

            @pl.when(ci + 2 < nch)
            def _():
                fetch_idx(ci + 2, u2)

        def group(gi, c):
            for u in range(4):
                slot(gi * 4 + u, u)
            return c
        lax.fori_loop(0, (nch + 3) // 4, group, 0)

        plsc.subcore_barrier()
        _dump_nd(s_sp, s_out, cid, sid, rps, rextra)

    fn = pl.kernel(
        body,
        out_type=jax.ShapeDtypeStruct((NC, n, 128), jnp.float32),
        mesh=_mesh(),
        compiler_params=pltpu.CompilerParams(needs_layout_passes=False),
        scratch_types=(
            [pltpu.VMEM((CHUNK,), jnp.int32) for _ in range(8)]
            + [pltpu.VMEM((CHUNK, 128), jnp.float32) for _ in range(2)]
            + [pltpu.VMEM_SHARED((n, 128), jnp.float32)]
            + [pltpu.SemaphoreType.DMA for _ in range(6)]
            + ([pltpu.VMEM((CHUNK,), jnp.float32) for _ in range(4)]
               + [pltpu.VMEM((CHUNK, 128), jnp.float32) for _ in range(2)]
               if scaled else [])
        ),
    )
    if scaled:
        return fn(h, src, dst, zeros_nd, w)
    return fn(h, src, dst, zeros_nd)


_ROWS = 2000


def _tc_qkvs(x, wc, bc, n):
    grid = n // _ROWS

    def body(x_ref, w_ref, b_ref, o_ref):
        o_ref[...] = jnp.dot(x_ref[...], w_ref[...],
                             preferred_element_type=jnp.float32) + b_ref[...]

    return pl.pallas_call(
        body,
        grid=(grid,),
        in_specs=[
            pl.BlockSpec((_ROWS, 128), lambda i: (i, 0)),
            pl.BlockSpec((128, 512), lambda i: (0, 0)),
            pl.BlockSpec((1, 512), lambda i: (0, 0)),
        ],
        out_specs=pl.BlockSpec((_ROWS, 512), lambda i: (i, 0)),
        out_shape=jax.ShapeDtypeStruct((n, 512), jnp.float32),
    )(x, wc, bc)


def _tc_h(zp, denp, s, n):
    grid = n // _ROWS

    def body(z_ref, d_ref, s_ref, o_ref):
        z = z_ref[0] + z_ref[1]
        den = jnp.sum(d_ref[0], axis=0) + 1e-16
        o_ref[...] = jnp.maximum(z / den[:, None] + s_ref[...], 0.0)

    return pl.pallas_call(
        body,
        grid=(grid,),
        in_specs=[
            pl.BlockSpec((NC, _ROWS, 128), lambda i: (0, i, 0)),
            pl.BlockSpec((1, NW, _ROWS), lambda i: (i, 0, 0)),
            pl.BlockSpec((_ROWS, 128), lambda i: (i, 0)),
        ],
        out_specs=pl.BlockSpec((_ROWS, 128), lambda i: (i, 0)),
        out_shape=jax.ShapeDtypeStruct((n, 128), jnp.float32),
    )(zp, denp, s)


def _tc_sage(sp, cntp, h, wl, wr, bf, n):
    grid = n // _ROWS

    def body(s_ref, c_ref, h_ref, wl_ref, wr_ref, b_ref, o_ref):
        cnt = jnp.maximum(jnp.sum(c_ref[0], axis=0), 1.0)
        mean = (s_ref[0] + s_ref[1]) / cnt[:, None]
        out = (jnp.dot(mean, wl_ref[...], preferred_element_type=jnp.float32)
               + jnp.dot(h_ref[...], wr_ref[...],
                         preferred_element_type=jnp.float32)
               + b_ref[...])
        o_ref[...] = jnp.maximum(out, 0.0)

    return pl.pallas_call(
        body,
        grid=(grid,),
        in_specs=[
            pl.BlockSpec((NC, _ROWS, 128), lambda i: (0, i, 0)),
            pl.BlockSpec((1, NW, _ROWS), lambda i: (i, 0, 0)),
            pl.BlockSpec((_ROWS, 128), lambda i: (i, 0)),
            pl.BlockSpec((128, 128), lambda i: (0, 0)),
            pl.BlockSpec((128, 128), lambda i: (0, 0)),
            pl.BlockSpec((1, 128), lambda i: (0, 0)),
        ],
        out_specs=pl.BlockSpec((_ROWS, 128), lambda i: (i, 0)),
        out_shape=jax.ShapeDtypeStruct((n, 128), jnp.float32),
    )(sp, cntp, h, wl, wr, bf)


def kernel(x, edge_index, params):
    n, d = x.shape
    e = edge_index.shape[1]
    assert d == 128

    src = edge_index[0].astype(jnp.int32)
    dst = edge_index[1].astype(jnp.int32)

    wc = jnp.concatenate([params["Wq"].T, params["Wk"].T,
                          params["Wv"].T, params["Wskip"].T], axis=1)
    bc = jnp.concatenate([params["bq"], params["bk"],
                          params["bv"], params["bskip"]])[None, :]
    a = jax.nn.sigmoid(params["res_alpha"])
    eye = jnp.eye(128, dtype=jnp.float32)
    wls, wrs, bfs = [], [], []
    for i in range(3):
        sbn = params["bn_g%d" % i] * lax.rsqrt(params["bn_v%d" % i] + 1e-5)
        wls.append(a * (params["W_l%d" % i].T * sbn[None, :]))
        wrs.append(a * (params["W_r%d" % i].T * sbn[None, :]) + (1.0 - a) * eye)
        bfs.append((a * ((params["b_l%d" % i] - params["bn_m%d" % i]) * sbn
                         + params["bn_b%d" % i]))[None, :])

    zeros_nd = jnp.zeros((n, 128), jnp.float32)

    qkvs = _tc_qkvs(x, wc, bc, n)
    q = qkvs[:, 0:128]
    k = qkvs[:, 128:256]
    v = qkvs[:, 256:384]
    s = qkvs[:, 384:512]

    we, denp, cntp = _sc_attn_w(q, k, src, dst, n, e)
    zp = _sc_scatter(v, src, dst, zeros_nd, n, e, w=we)

    grid = n // _ROWS
    denp = denp.reshape(NW, grid, _ROWS).transpose(1, 0, 2)
    cntp = cntp.reshape(NW, grid, _ROWS).transpose(1, 0, 2)

    h = _tc_h(zp, denp, s, n)
    for i in range(3):
        sp = _sc_scatter(h, src, dst, zeros_nd, n, e)
        h = _tc_sage(sp, cntp, h, wls[i], wrs[i], bfs[i], n)
    return h


# speedup vs baseline: 12.3883x; 1.3425x over previous
import jax
import jax.numpy as jnp
from jax import lax
from jax.experimental import pallas as pl
from jax.experimental.pallas import tpu as pltpu
from jax.experimental.pallas import tpu_sc as plsc

NC = 2
NS = 16
NW = NC * NS
LANES = 16
CHUNK = 80

_INV_SCALE = 1.0 / (128.0 ** 0.5)


def _mesh():
    return plsc.VectorSubcoreMesh(
        core_axis_name="c", subcore_axis_name="s", num_cores=NC, num_subcores=NS)


def _wid():
    return lax.axis_index("s") * NC + lax.axis_index("c")


def _zero_nd(zf_hbm, acc_sp, sid, rps, rextra):
    pltpu.sync_copy(zf_hbm.at[pl.ds(sid * rps, rps)],
                    acc_sp.at[pl.ds(sid * rps, rps)])
    if rextra:
        @pl.when(sid == NS - 1)
        def _():
            pltpu.sync_copy(zf_hbm.at[pl.ds(NS * rps, rextra)],
                            acc_sp.at[pl.ds(NS * rps, rextra)])


def _dump_nd(acc_sp, out, cid, sid, rps, rextra):
    pltpu.sync_copy(acc_sp.at[pl.ds(sid * rps, rps)],
                    out.at[cid, pl.ds(sid * rps, rps)])
    if rextra:
        @pl.when(sid == NS - 1)
        def _():
            pltpu.sync_copy(acc_sp.at[pl.ds(NS * rps, rextra)],
                            out.at[cid, pl.ds(NS * rps, rextra)])


def _sc_attn_w(q, k, src, dst, n, e):
    ep = e // NW
    nch = ep // CHUNK
    assert ep % CHUNK == 0 and CHUNK % LANES == 0

    def body(q_hbm, k_hbm, src_hbm, dst_hbm,
             w_out, den_out, cnt_out, *scr):
        srcv = list(scr[0:4])
        dstv = list(scr[4:8])
        qb = list(scr[8:10])
        kb = list(scr[10:12])
        wb = list(scr[12:14])
        den_l, cnt_l = scr[14], scr[15]
        semi = list(scr[16:20])
        semg = list(scr[20:22])
        semw = list(scr[22:24])

        wid = _wid()
        base0 = wid * ep

        def zloc(i, c):
            zv = jnp.zeros((LANES,), jnp.float32)
            den_l[pl.ds(i * LANES, LANES)] = zv
            cnt_l[pl.ds(i * LANES, LANES)] = zv
            return c
        lax.fori_loop(0, n // LANES, zloc, 0)

        pltpu.async_copy(src_hbm.at[pl.ds(base0, CHUNK)], srcv[0], semi[0])
        pltpu.async_copy(dst_hbm.at[pl.ds(base0, CHUNK)], dstv[0], semi[0])
        pltpu.async_copy(src_hbm.at[pl.ds(base0 + CHUNK, CHUNK)], srcv[1],
                         semi[1])
        pltpu.async_copy(dst_hbm.at[pl.ds(base0 + CHUNK, CHUNK)], dstv[1],
                         semi[1])
        pltpu.make_async_copy(src_hbm.at[pl.ds(base0, CHUNK)], srcv[0],
                              semi[0]).wait()
        pltpu.make_async_copy(dst_hbm.at[pl.ds(base0, CHUNK)], dstv[0],
                              semi[0]).wait()
        pltpu.async_copy(q_hbm.at[dstv[0]], qb[0], semg[0])
        pltpu.async_copy(k_hbm.at[srcv[0]], kb[0], semg[0])

        lanes = lax.iota(jnp.int32, LANES)

        def slot(ci, u):
            b = u % 2
            u1, b1 = (u + 1) % 4, (u + 1) % 2
            u2 = (u + 2) % 4
            live = ci < nch
            base = base0 + ci * CHUNK

            @pl.when(live)
            def _():
                pltpu.make_async_copy(q_hbm.at[dstv[u]], qb[b],
                                      semg[b]).wait()
                pltpu.make_async_copy(k_hbm.at[srcv[u]], kb[b],
                                      semg[b]).wait()

            @pl.when(jnp.logical_and(live, ci >= 2))
            def _():
                basep = base0 + (ci - 2) * CHUNK
                pltpu.make_async_copy(wb[b], w_out.at[pl.ds(basep, CHUNK)],
                                      semw[b]).wait()

            @pl.when(ci + 1 < nch)
            def _():
                base1 = base0 + (ci + 1) * CHUNK
                pltpu.make_async_copy(src_hbm.at[pl.ds(base1, CHUNK)],
                                      srcv[u1], semi[u1]).wait()
                pltpu.make_async_copy(dst_hbm.at[pl.ds(base1, CHUNK)],
                                      dstv[u1], semi[u1]).wait()
                pltpu.async_copy(q_hbm.at[dstv[u1]], qb[b1], semg[b1])
                pltpu.async_copy(k_hbm.at[srcv[u1]], kb[b1], semg[b1])

            @pl.when(live)
            def _():
                ones16 = jnp.ones((LANES,), jnp.float32)
                for g in range(CHUNK // LANES):
                    rows = g * LANES + lanes
                    dstg = dstv[u][pl.ds(g * LANES, LANES)]

                    def dot_body(dd, acc):
                        for uu in range(4):
                            col = (lanes + (dd * 4 + uu)) & 127
                            qv = plsc.load_gather(qb[b], [rows, col])
                            kv = plsc.load_gather(kb[b], [rows, col])
                            acc = acc + qv * kv
                        return acc
                    acc = lax.fori_loop(0, 32, dot_body,
                                        jnp.zeros((LANES,), jnp.float32))
                    w16 = jnp.exp(acc * _INV_SCALE)
                    wb[b][pl.ds(g * LANES, LANES)] = w16
                    plsc.addupdate_scatter(den_l, [dstg], w16)
                    plsc.addupdate_scatter(cnt_l, [dstg], ones16)
                pltpu.async_copy(wb[b], w_out.at[pl.ds(base, CHUNK)], semw[b])

            @pl.when(ci + 2 < nch)
            def _():
                base2 = base0 + (ci + 2) * CHUNK
                pltpu.async_copy(src_hbm.at[pl.ds(base2, CHUNK)], srcv[u2],
                                 semi[u2])
                pltpu.async_copy(dst_hbm.at[pl.ds(base2, CHUNK)], dstv[u2],
                                 semi[u2])

        def group(gi, c):
            for u in range(4):
                slot(gi * 4 + u, u)
            return c
        lax.fori_loop(0, (nch + 3) // 4, group, 0)

        for cl in (nch - 2, nch - 1):
            basel = base0 + cl * CHUNK
            pltpu.make_async_copy(wb[cl % 2], w_out.at[pl.ds(basel, CHUNK)],
                                  semw[cl % 2]).wait()
        pltpu.sync_copy(den_l, den_out.at[pl.ds(wid * n, n)])
        pltpu.sync_copy(cnt_l, cnt_out.at[pl.ds(wid * n, n)])

    fn = pl.kernel(
        body,
        out_type=(jax.ShapeDtypeStruct((e,), jnp.float32),
                  jax.ShapeDtypeStruct((NW * n,), jnp.float32),
                  jax.ShapeDtypeStruct((NW * n,), jnp.float32)),
        mesh=_mesh(),
        compiler_params=pltpu.CompilerParams(needs_layout_passes=False),
        scratch_types=(
            [pltpu.VMEM((CHUNK,), jnp.int32) for _ in range(8)]
            + [pltpu.VMEM((CHUNK, 128), jnp.float32) for _ in range(4)]
            + [pltpu.VMEM((CHUNK,), jnp.float32) for _ in range(2)]
            + [pltpu.VMEM((n,), jnp.float32) for _ in range(2)]
            + [pltpu.SemaphoreType.DMA for _ in range(8)]
        ),
    )
    return fn(q, k, src, dst)


def _sc_scatter(h, src, dst, zeros_nd, n, e, w=None):
    ep = e // NW
    nch = ep // CHUNK
    rps = (n // NS) & ~7
    rextra = n - NS * rps
    scaled = w is not None

    def body(*args):
        if scaled:
            (h_hbm, src_hbm, dst_hbm, zf_hbm, w_hbm, s_out, *scr) = args
        else:
            (h_hbm, src_hbm, dst_hbm, zf_hbm, s_out, *scr) = args
        srcv = list(scr[0:4])
        dstv = list(scr[4:8])
        buf = list(scr[8:10])
        s_sp = scr[10]
        semi = list(scr[11:15])
        semg = list(scr[15:17])
        wv = list(scr[17:21]) if scaled else None
        buf2 = list(scr[21:23]) if scaled else buf

        cid = lax.axis_index("c")
        sid = lax.axis_index("s")
        wid = _wid()
        base0 = wid * ep

        _zero_nd(zf_hbm, s_sp, sid, rps, rextra)
        plsc.subcore_barrier()

        def fetch_idx(ci, u):
            base = base0 + ci * CHUNK
            pltpu.async_copy(src_hbm.at[pl.ds(base, CHUNK)], srcv[u], semi[u])
            pltpu.async_copy(dst_hbm.at[pl.ds(base, CHUNK)], dstv[u], semi[u])
            if scaled:
                pltpu.async_copy(w_hbm.at[pl.ds(base, CHUNK)], wv[u], semi[u])

        def wait_idx(ci, u):
            base = base0 + ci * CHUNK
            pltpu.make_async_copy(src_hbm.at[pl.ds(base, CHUNK)], srcv[u],
                                  semi[u]).wait()
            pltpu.make_async_copy(dst_hbm.at[pl.ds(base, CHUNK)], dstv[u],
                                  semi[u]).wait()
            if scaled:
                pltpu.make_async_copy(w_hbm.at[pl.ds(base, CHUNK)], wv[u],
                                      semi[u]).wait()

        fetch_idx(0, 0)
        fetch_idx(1, 1)
        wait_idx(0, 0)
        pltpu.async_copy(h_hbm.at[srcv[0]], buf[0], semg[0])

        lanes = lax.iota(jnp.int32, LANES)

        def slot(ci, u):
            b = u % 2
            u1, b1 = (u + 1) % 4, (u + 1) % 2
            u2 = (u + 2) % 4
            live = ci < nch

            @pl.when(live)
            def _():
                pltpu.make_async_copy(h_hbm.at[srcv[u]], buf[b],
                                      semg[b]).wait()

            @pl.when(ci + 1 < nch)
            def _():
                wait_idx(ci + 1, u1)
                pltpu.async_copy(h_hbm.at[srcv[u1]], buf[b1], semg[b1])

            @pl.when(live)
            def _():
                if scaled:
                    @plsc.parallel_loop(0, CHUNK, step=1, unroll=4)
                    def _(ei):
                        wsp = plsc.load_gather(
                            wv[u], [jnp.full((LANES,), ei, jnp.int32)])
                        for j in range(8):
                            seg = buf[b][ei, pl.ds(j * LANES, LANES)]
                            buf2[b][ei, pl.ds(j * LANES, LANES)] = seg * wsp
                pltpu.sync_copy(buf2[b], s_sp.at[dstv[u]], add=True)

            @pl.when(ci + 2 < nch)
            def _():
                fetch_idx(ci + 2, u2)

        def group(gi, c):
            for u in range(4):
                slot(gi * 4 + u, u)
            return c
        lax.fori_loop(0, (nch + 3) // 4, group, 0)

        plsc.subcore_barrier()
        _dump_nd(s_sp, s_out, cid, sid, rps, rextra)

    fn = pl.kernel(
        body,
        out_type=jax.ShapeDtypeStruct((NC, n, 128), jnp.float32),
        mesh=_mesh(),
        compiler_params=pltpu.CompilerParams(needs_layout_passes=False),
        scratch_types=(
            [pltpu.VMEM((CHUNK,), jnp.int32) for _ in range(8)]
            + [pltpu.VMEM((CHUNK, 128), jnp.float32) for _ in range(2)]
            + [pltpu.VMEM_SHARED((n, 128), jnp.float32)]
            + [pltpu.SemaphoreType.DMA for _ in range(6)]
            + ([pltpu.VMEM((CHUNK,), jnp.float32) for _ in range(4)]
               + [pltpu.VMEM((CHUNK, 128), jnp.float32) for _ in range(2)]
               if scaled else [])
        ),
    )
    if scaled:
        return fn(h, src, dst, zeros_nd, w)
    return fn(h, src, dst, zeros_nd)


_ROWS = 2000


def _tc_qkvs(x, wc, bc, n):
    grid = n // _ROWS

    def body(x_ref, w_ref, b_ref, o_ref):
        o_ref[...] = jnp.dot(x_ref[...], w_ref[...],
                             preferred_element_type=jnp.float32) + b_ref[...]

    return pl.pallas_call(
        body,
        grid=(grid,),
        in_specs=[
            pl.BlockSpec((_ROWS, 128), lambda i: (i, 0)),
            pl.BlockSpec((128, 512), lambda i: (0, 0)),
            pl.BlockSpec((1, 512), lambda i: (0, 0)),
        ],
        out_specs=pl.BlockSpec((_ROWS, 512), lambda i: (i, 0)),
        out_shape=jax.ShapeDtypeStruct((n, 512), jnp.float32),
    )(x, wc, bc)


def _tc_h(zp, denp, s, n):
    grid = n // _ROWS

    def body(z_ref, d_ref, s_ref, o_ref):
        z = z_ref[0] + z_ref[1]
        den = jnp.sum(d_ref[0], axis=0) + 1e-16
        o_ref[...] = jnp.maximum(z / den[:, None] + s_ref[...], 0.0)

    return pl.pallas_call(
        body,
        grid=(grid,),
        in_specs=[
            pl.BlockSpec((NC, _ROWS, 128), lambda i: (0, i, 0)),
            pl.BlockSpec((1, NW, _ROWS), lambda i: (i, 0, 0)),
            pl.BlockSpec((_ROWS, 128), lambda i: (i, 0)),
        ],
        out_specs=pl.BlockSpec((_ROWS, 128), lambda i: (i, 0)),
        out_shape=jax.ShapeDtypeStruct((n, 128), jnp.float32),
    )(zp, denp, s)


def _tc_sage(sp, cntp, h, wl, wr, bf, n):
    grid = n // _ROWS

    def body(s_ref, c_ref, h_ref, wl_ref, wr_ref, b_ref, o_ref):
        cnt = jnp.maximum(jnp.sum(c_ref[0], axis=0), 1.0)
        mean = (s_ref[0] + s_ref[1]) / cnt[:, None]
        out = (jnp.dot(mean, wl_ref[...], preferred_element_type=jnp.float32)
               + jnp.dot(h_ref[...], wr_ref[...],
                         preferred_element_type=jnp.float32)
               + b_ref[...])
        o_ref[...] = jnp.maximum(out, 0.0)

    return pl.pallas_call(
        body,
        grid=(grid,),
        in_specs=[
            pl.BlockSpec((NC, _ROWS, 128), lambda i: (0, i, 0)),
            pl.BlockSpec((1, NW, _ROWS), lambda i: (i, 0, 0)),
            pl.BlockSpec((_ROWS, 128), lambda i: (i, 0)),
            pl.BlockSpec((128, 128), lambda i: (0, 0)),
            pl.BlockSpec((128, 128), lambda i: (0, 0)),
            pl.BlockSpec((1, 128), lambda i: (0, 0)),
        ],
        out_specs=pl.BlockSpec((_ROWS, 128), lambda i: (i, 0)),
        out_shape=jax.ShapeDtypeStruct((n, 128), jnp.float32),
    )(sp, cntp, h, wl, wr, bf)


def kernel(x, edge_index, params):
    n, d = x.shape
    e = edge_index.shape[1]
    assert d == 128

    src = edge_index[0].astype(jnp.int32)
    dst = edge_index[1].astype(jnp.int32)

    wc = jnp.concatenate([params["Wq"].T, params["Wk"].T,
                          params["Wv"].T, params["Wskip"].T], axis=1)
    bc = jnp.concatenate([params["bq"], params["bk"],
                          params["bv"], params["bskip"]])[None, :]
    a = jax.nn.sigmoid(params["res_alpha"])
    eye = jnp.eye(128, dtype=jnp.float32)
    wls, wrs, bfs = [], [], []
    for i in range(3):
        sbn = params["bn_g%d" % i] * lax.rsqrt(params["bn_v%d" % i] + 1e-5)
        wls.append(a * (params["W_l%d" % i].T * sbn[None, :]))
        wrs.append(a * (params["W_r%d" % i].T * sbn[None, :]) + (1.0 - a) * eye)
        bfs.append((a * ((params["b_l%d" % i] - params["bn_m%d" % i]) * sbn
                         + params["bn_b%d" % i]))[None, :])

    zeros_nd = jnp.zeros((n, 128), jnp.float32)

    qkvs = _tc_qkvs(x, wc, bc, n)
    q = qkvs[:, 0:128]
    k = qkvs[:, 128:256]
    v = qkvs[:, 256:384]
    s = qkvs[:, 384:512]

    we, denp, cntp = _sc_attn_w(q, k, src, dst, n, e)
    zp = _sc_scatter(v, src, dst, zeros_nd, n, e, w=we)

    grid = n // _ROWS
    denp = denp.reshape(NW, grid, _ROWS).transpose(1, 0, 2)
    cntp = cntp.reshape(NW, grid, _ROWS).transpose(1, 0, 2)

    h = _tc_h(zp, denp, s, n)
    for i in range(3):
        sp = _sc_scatter(h, src, dst, zeros_nd, n, e)
        h = _tc_sage(sp, cntp, h, wls[i], wrs[i], bfs[i], n)
    return h
